# baseline (speedup 1.0000x reference)
import functools

import jax
import jax.numpy as jnp
from jax.experimental import pallas as pl
from jax.experimental.pallas import tpu as pltpu

NW, NT, ND, D = 100000, 5000, 20000, 128
_BR = 2000


def _word_proj_body(x_ref, www_ref, wwt_ref, wwd_ref, o1_ref, o2_ref, o3_ref):
    x = x_ref[...]
    o1_ref[...] = jnp.dot(x, www_ref[...], preferred_element_type=jnp.float32)
    o2_ref[...] = jnp.dot(x, wwt_ref[...], preferred_element_type=jnp.float32)
    o3_ref[...] = jnp.dot(x, wwd_ref[...], preferred_element_type=jnp.float32)


def _topic_proj_body(ft_ref, eff_ref, wtd_ref, wtt_ref, wcau_ref, wnoi_ref,
                     otd_ref, ott_ref):
    ft = ft_ref[...]
    eff = eff_ref[...]
    pos = (eff > 0).astype(jnp.float32)
    neg = (eff < 0).astype(jnp.float32)
    cau = jnp.dot(ft * pos, wcau_ref[...], preferred_element_type=jnp.float32)
    noi = jnp.dot(ft * neg, wnoi_ref[...], preferred_element_type=jnp.float32)
    cmn = cau - noi
    otd_ref[...] = jnp.dot(ft, wtd_ref[...], preferred_element_type=jnp.float32) + cmn
    ott_ref[...] = jnp.dot(ft, wtt_ref[...], preferred_element_type=jnp.float32) + cmn


def _word_proj(feat_word, W_ww, W_wt, W_wd):
    n = feat_word.shape[0]
    grid = n // _BR
    blk = pl.BlockSpec((_BR, D), lambda i: (i, 0))
    wblk = pl.BlockSpec((D, D), lambda i: (0, 0))
    return pl.pallas_call(
        _word_proj_body,
        grid=(grid,),
        in_specs=[blk, wblk, wblk, wblk],
        out_specs=[blk, blk, blk],
        out_shape=[jax.ShapeDtypeStruct((n, D), jnp.float32)] * 3,
    )(feat_word, W_ww, W_wt, W_wd)


def _topic_proj(feat_topic, effect, W_td, W_tt, W_cau, W_noi):
    n = feat_topic.shape[0]
    return pl.pallas_call(
        _topic_proj_body,
        out_shape=[jax.ShapeDtypeStruct((n, D), jnp.float32)] * 2,
    )(feat_topic, effect, W_td, W_tt, W_cau, W_noi)


def _mean_agg(Wh, src, dst, w, num_dst):
    m = jnp.take(Wh, src, axis=0) * w[:, None]
    s = jax.ops.segment_sum(m, dst, num_segments=num_dst)
    cnt = jax.ops.segment_sum(jnp.ones_like(w), dst, num_segments=num_dst)
    return jnp.where(cnt[:, None] > 0, s / jnp.maximum(cnt, 1.0)[:, None], 0.0)


def kernel(feat_word, feat_topic, effect, ww_w, wt_w, wd_w, td_w, tt_w,
           W_ww, b_ww, W_wt, b_wt, W_wd, b_wd, W_td, b_td, W_tt, b_tt,
           W_cau, W_noi,
           ww_src, ww_dst, wt_src, wt_dst, wd_src, wd_dst,
           td_src, td_dst, tt_src, tt_dst):
    Wh_ww, Wh_wt, Wh_wd = _word_proj(feat_word, W_ww, W_wt, W_wd)
    Wh_ww = Wh_ww + b_ww
    Wh_wt = Wh_wt + b_wt
    Wh_wd = Wh_wd + b_wd
    Wh_td, Wh_tt = _topic_proj(feat_topic, effect, W_td, W_tt, W_cau, W_noi)
    Wh_td = Wh_td + b_td
    Wh_tt = Wh_tt + b_tt
    h_word = _mean_agg(Wh_ww, ww_src, ww_dst, ww_w, NW)
    h_topic = _mean_agg(Wh_wt, wt_src, wt_dst, wt_w, NT) + _mean_agg(Wh_tt, tt_src, tt_dst, tt_w, NT)
    h_doc = _mean_agg(Wh_wd, wd_src, wd_dst, wd_w, ND) + _mean_agg(Wh_td, td_src, td_dst, td_w, ND)
    return jnp.concatenate([h_word, h_topic, h_doc], axis=0)


# speedup vs baseline: 1.8202x; 1.8202x over previous
import functools

import jax
import jax.numpy as jnp
from jax import lax
from jax.experimental import pallas as pl
from jax.experimental.pallas import tpu as pltpu
from jax.experimental.pallas import tpu_sc as plsc

NW, NT, ND, D = 100000, 5000, 20000, 128
_BR = 2000
_C = 12288
_SB = 2048
_STG = 160
_NSUB = 16

_NCH_W = -(-NW // _C)
_NCH_T = -(-NT // _C)
_NCH_D = -(-ND // _C)
_NWP = _NCH_W * _C
_NTP = _NCH_T * _C
_NDP = _NCH_D * _C


def _word_proj_body(x_ref, www_ref, wwt_ref, wwd_ref, bww_ref, bwt_ref,
                    bwd_ref, o1_ref, o2_ref, o3_ref):
    x = x_ref[...]
    o1_ref[...] = jnp.dot(x, www_ref[...], preferred_element_type=jnp.float32) + bww_ref[...]
    o2_ref[...] = jnp.dot(x, wwt_ref[...], preferred_element_type=jnp.float32) + bwt_ref[...]
    o3_ref[...] = jnp.dot(x, wwd_ref[...], preferred_element_type=jnp.float32) + bwd_ref[...]


def _topic_proj_body(ft_ref, eff_ref, wtd_ref, wtt_ref, wcau_ref, wnoi_ref,
                     btd_ref, btt_ref, otd_ref, ott_ref):
    ft = ft_ref[...]
    eff = eff_ref[...]
    pos = (eff > 0).astype(jnp.float32)
    neg = (eff < 0).astype(jnp.float32)
    cau = jnp.dot(ft * pos, wcau_ref[...], preferred_element_type=jnp.float32)
    noi = jnp.dot(ft * neg, wnoi_ref[...], preferred_element_type=jnp.float32)
    cmn = cau - noi
    otd_ref[...] = jnp.dot(ft, wtd_ref[...], preferred_element_type=jnp.float32) + btd_ref[...] + cmn
    ott_ref[...] = jnp.dot(ft, wtt_ref[...], preferred_element_type=jnp.float32) + btt_ref[...] + cmn


def _word_proj(feat_word, W_ww, W_wt, W_wd, b_ww, b_wt, b_wd):
    n = feat_word.shape[0]
    blk = pl.BlockSpec((_BR, D), lambda i: (i, 0))
    wblk = pl.BlockSpec((D, D), lambda i: (0, 0))
    bblk = pl.BlockSpec((1, D), lambda i: (0, 0))
    return pl.pallas_call(
        _word_proj_body,
        grid=(n // _BR,),
        in_specs=[blk, wblk, wblk, wblk, bblk, bblk, bblk],
        out_specs=[blk, blk, blk],
        out_shape=[jax.ShapeDtypeStruct((n, D), jnp.float32)] * 3,
    )(feat_word, W_ww, W_wt, W_wd, b_ww.reshape(1, D), b_wt.reshape(1, D),
      b_wd.reshape(1, D))


def _topic_proj(feat_topic, effect, W_td, W_tt, W_cau, W_noi, b_td, b_tt):
    n = feat_topic.shape[0]
    return pl.pallas_call(
        _topic_proj_body,
        out_shape=[jax.ShapeDtypeStruct((n, D), jnp.float32)] * 2,
    )(feat_topic, effect, W_td, W_tt, W_cau, W_noi, b_td.reshape(1, D),
      b_tt.reshape(1, D))


def _pad_edges(src, dst, w):
    e = src.shape[0]
    e_pad = _NSUB * _SB * (-(-e // (_NSUB * _SB)))
    pad = e_pad - e
    return (jnp.pad(src, (0, pad)),
            jnp.pad(dst, (0, pad), constant_values=-1),
            jnp.pad(w, (0, pad)))


def _iota16():
    return lax.iota(jnp.int32, 16)


def _splat16(x):
    return jnp.full((16,), x, jnp.int32)


def _sc_agg_body(
    wh_ww, wh_wt, wh_wd, wh_td, wh_tt,
    ww_src, ww_dst, ww_w, wt_src, wt_dst, wt_w, wd_src, wd_dst, wd_w,
    td_src, td_dst, td_w, tt_src, tt_dst, tt_w,
    out_w, out_t, out_d,
    feat_sh, cnt_sh,
    scan_src, scan_dst, scan_w,
    stage_src, stage_dloc, stage_w,
    dma_src, dma_dloc, dma_w,
    rows, cbuf, ones_v, zc_v,
):
    cid = lax.axis_index("c")
    sid = lax.axis_index("s")

    for j in range(8):
        ones_v[pl.ds(j * 16, 16)] = jnp.ones((16,), jnp.float32)
        zc_v[pl.ds(j * 16, 16)] = jnp.zeros((16,), jnp.float32)

    rows_per_tile = _C // _NSUB
    tile_row0 = sid * rows_per_tile

    def _flush(wh_hbm):
        pltpu.sync_copy(wh_hbm.at[dma_src], rows)

        def _scale(r, _):
            wv = plsc.load_gather(dma_w, [_splat16(r)])
            for j in range(8):
                sl = pl.ds(j * 16, 16)
                rows[r, sl] = rows[r, sl] * wv
            return 0
        lax.fori_loop(0, 128, _scale, 0)
        pltpu.sync_copy(rows, feat_sh.at[dma_dloc], add=True)
        pltpu.sync_copy(ones_v, cnt_sh.at[dma_dloc], add=True)

    def _pass(wh_hbm, src_hbm, dst_hbm, w_hbm, e_pad, out_hbm, lo, accumulate):
        def _zrow(r, _):
            for j in range(8):
                rows[r, pl.ds(j * 16, 16)] = jnp.zeros((16,), jnp.float32)
            return 0
        lax.fori_loop(0, 128, _zrow, 0)
        for b in range(rows_per_tile // 128):
            pltpu.sync_copy(rows, feat_sh.at[pl.ds(tile_row0 + b * 128, 128)])
            pltpu.sync_copy(zc_v, cnt_sh.at[pl.ds(tile_row0 + b * 128, 128)])
        plsc.subcore_barrier()

        pt = e_pad // _NSUB
        e0 = sid * pt
        hi = lo + _C

        def _scan_block(bi, k):
            pos0 = e0 + bi * _SB
            pltpu.sync_copy(dst_hbm.at[pl.ds(pos0, _SB)], scan_dst)
            pltpu.sync_copy(src_hbm.at[pl.ds(pos0, _SB)], scan_src)
            pltpu.sync_copy(w_hbm.at[pl.ds(pos0, _SB)], scan_w)

            def _step(i, k):
                sl = pl.ds(i * 16, 16)
                dstv = scan_dst[sl]
                inr = (dstv >= lo) & (dstv < hi)
                inr32 = jnp.where(inr, _splat16(1), _splat16(0))
                ps = plsc.cumsum(inr32)
                pos = jnp.where(inr, _splat16(k) + ps - 1, _splat16(_STG - 8))
                plsc.store_scatter(stage_dloc, [pos], dstv - lo)
                plsc.store_scatter(stage_src, [pos], scan_src[sl])
                plsc.store_scatter(stage_w, [pos], scan_w[sl])
                k = k + jnp.sum(inr32)

                def _full(k):
                    for j in range(8):
                        s2 = pl.ds(j * 16, 16)
                        dma_dloc[s2] = stage_dloc[s2]
                        dma_src[s2] = stage_src[s2]
                        dma_w[s2] = stage_w[s2]
                    _flush(wh_hbm)
                    stage_dloc[pl.ds(0, 16)] = stage_dloc[pl.ds(128, 16)]
                    stage_src[pl.ds(0, 16)] = stage_src[pl.ds(128, 16)]
                    stage_w[pl.ds(0, 16)] = stage_w[pl.ds(128, 16)]
                    return k - 128

                return lax.cond(k >= 128, _full, lambda k: k, k)

            return lax.fori_loop(0, _SB // 16, _step, k)

        k = lax.fori_loop(0, pt // _SB, _scan_block, 0)

        for j in range(8):
            s2 = pl.ds(j * 16, 16)
            lanes = _splat16(j * 16) + _iota16()
            live = lanes < _splat16(k)
            dma_dloc[s2] = jnp.where(live, stage_dloc[s2], _splat16(_C))
            dma_src[s2] = jnp.where(live, stage_src[s2], _splat16(0))
            dma_w[s2] = stage_w[s2]
        _flush(wh_hbm)
        plsc.subcore_barrier()

        def _finblk(b, _):
            row0 = tile_row0 + b * 64
            pltpu.sync_copy(feat_sh.at[pl.ds(row0, 64)], rows.at[pl.ds(0, 64)])
            pltpu.sync_copy(cnt_sh.at[pl.ds(row0, 64)], cbuf.at[pl.ds(0, 64)])
            if accumulate:
                pltpu.sync_copy(out_hbm.at[pl.ds(lo + row0, 64)],
                                rows.at[pl.ds(64, 64)])

            def _fin(r, _):
                c = plsc.load_gather(cbuf, [_splat16(r)])
                recip = jnp.where(c > 0, 1.0 / jnp.maximum(c, 1.0), 0.0)
                for j in range(8):
                    sl = pl.ds(j * 16, 16)
                    if accumulate:
                        rows[r, sl] = rows[r, sl] * recip + rows[r + 64, sl]
                    else:
                        rows[r, sl] = rows[r, sl] * recip
                return 0
            lax.fori_loop(0, 64, _fin, 0)
            pltpu.sync_copy(rows.at[pl.ds(0, 64)],
                            out_hbm.at[pl.ds(lo + row0, 64)])
            return 0
        lax.fori_loop(0, rows_per_tile // 64, _finblk, 0)
        plsc.subcore_barrier()

    def _etype(wh_hbm, src_hbm, dst_hbm, w_hbm, e_pad, out_hbm, nch,
               accumulate):
        trips = -(-nch // 2)

        def _chunk(ci, _):
            chunk = 2 * ci + cid

            @pl.when(chunk < nch)
            def _():
                _pass(wh_hbm, src_hbm, dst_hbm, w_hbm, e_pad, out_hbm,
                      chunk * _C, accumulate)
            return 0
        lax.fori_loop(0, trips, _chunk, 0)

    _etype(wh_ww, ww_src, ww_dst, ww_w, ww_src.shape[0], out_w, _NCH_W, False)
    _etype(wh_wt, wt_src, wt_dst, wt_w, wt_src.shape[0], out_t, _NCH_T, False)
    _etype(wh_tt, tt_src, tt_dst, tt_w, tt_src.shape[0], out_t, _NCH_T, True)
    _etype(wh_wd, wd_src, wd_dst, wd_w, wd_src.shape[0], out_d, _NCH_D, False)
    _etype(wh_td, td_src, td_dst, td_w, td_src.shape[0], out_d, _NCH_D, True)


def _sc_aggregate(wh_ww, wh_wt, wh_wd, wh_td, wh_tt, edges):
    (ww_src, ww_dst, ww_w, wt_src, wt_dst, wt_w, wd_src, wd_dst, wd_w,
     td_src, td_dst, td_w, tt_src, tt_dst, tt_w) = edges
    f32 = jnp.float32
    mesh = plsc.VectorSubcoreMesh(core_axis_name="c", subcore_axis_name="s")
    run = pl.kernel(
        _sc_agg_body,
        mesh=mesh,
        compiler_params=pltpu.CompilerParams(needs_layout_passes=False),
        out_type=[
            jax.ShapeDtypeStruct((_NWP, D), f32),
            jax.ShapeDtypeStruct((_NTP, D), f32),
            jax.ShapeDtypeStruct((_NDP, D), f32),
        ],
        scratch_types=[
            pltpu.VMEM_SHARED((_C + 8, D), f32),
            pltpu.VMEM_SHARED((_C + 8,), f32),
            pltpu.VMEM((_SB,), jnp.int32),
            pltpu.VMEM((_SB,), jnp.int32),
            pltpu.VMEM((_SB,), f32),
            pltpu.VMEM((_STG,), jnp.int32),
            pltpu.VMEM((_STG,), jnp.int32),
            pltpu.VMEM((_STG,), f32),
            pltpu.VMEM((128,), jnp.int32),
            pltpu.VMEM((128,), jnp.int32),
            pltpu.VMEM((128,), f32),
            pltpu.VMEM((128, D), f32),
            pltpu.VMEM((128,), f32),
            pltpu.VMEM((128,), f32),
            pltpu.VMEM((128,), f32),
        ],
    )
    return run(wh_ww, wh_wt, wh_wd, wh_td, wh_tt,
               ww_src, ww_dst, ww_w, wt_src, wt_dst, wt_w,
               wd_src, wd_dst, wd_w, td_src, td_dst, td_w,
               tt_src, tt_dst, tt_w)


def kernel(feat_word, feat_topic, effect, ww_w, wt_w, wd_w, td_w, tt_w,
           W_ww, b_ww, W_wt, b_wt, W_wd, b_wd, W_td, b_td, W_tt, b_tt,
           W_cau, W_noi,
           ww_src, ww_dst, wt_src, wt_dst, wd_src, wd_dst,
           td_src, td_dst, tt_src, tt_dst):
    wh_ww, wh_wt, wh_wd = _word_proj(feat_word, W_ww, W_wt, W_wd,
                                     b_ww, b_wt, b_wd)
    wh_td, wh_tt = _topic_proj(feat_topic, effect, W_td, W_tt, W_cau, W_noi,
                               b_td, b_tt)
    flat = []
    for t in (_pad_edges(ww_src, ww_dst, ww_w),
              _pad_edges(wt_src, wt_dst, wt_w),
              _pad_edges(wd_src, wd_dst, wd_w),
              _pad_edges(td_src, td_dst, td_w),
              _pad_edges(tt_src, tt_dst, tt_w)):
        flat.extend(t)
    h_word, h_topic, h_doc = _sc_aggregate(wh_ww, wh_wt, wh_wd, wh_td, wh_tt,
                                           tuple(flat))
    return jnp.concatenate(
        [h_word[:NW], h_topic[:NT], h_doc[:ND]], axis=0)


# speedup vs baseline: 2.2887x; 1.2574x over previous
import functools

import jax
import jax.numpy as jnp
from jax import lax
from jax.experimental import pallas as pl
from jax.experimental.pallas import tpu as pltpu
from jax.experimental.pallas import tpu_sc as plsc

NW, NT, ND, D = 100000, 5000, 20000, 128
_BR = 2000
_C = 10240
_SB = 2048
_STG = 160
_NSUB = 16

_NCH_W = -(-NW // _C)
_NCH_T = -(-NT // _C)
_NCH_D = -(-ND // _C)
_NWP = _NCH_W * _C
_NTP = _NCH_T * _C
_NDP = _NCH_D * _C


def _word_proj_body(x_ref, www_ref, wwt_ref, wwd_ref, bww_ref, bwt_ref,
                    bwd_ref, o1_ref, o2_ref, o3_ref):
    x = x_ref[...]
    o1_ref[...] = jnp.dot(x, www_ref[...], preferred_element_type=jnp.float32) + bww_ref[...]
    o2_ref[...] = jnp.dot(x, wwt_ref[...], preferred_element_type=jnp.float32) + bwt_ref[...]
    o3_ref[...] = jnp.dot(x, wwd_ref[...], preferred_element_type=jnp.float32) + bwd_ref[...]


def _topic_proj_body(ft_ref, eff_ref, wtd_ref, wtt_ref, wcau_ref, wnoi_ref,
                     btd_ref, btt_ref, otd_ref, ott_ref):
    ft = ft_ref[...]
    eff = eff_ref[...]
    pos = (eff > 0).astype(jnp.float32)
    neg = (eff < 0).astype(jnp.float32)
    cau = jnp.dot(ft * pos, wcau_ref[...], preferred_element_type=jnp.float32)
    noi = jnp.dot(ft * neg, wnoi_ref[...], preferred_element_type=jnp.float32)
    cmn = cau - noi
    otd_ref[...] = jnp.dot(ft, wtd_ref[...], preferred_element_type=jnp.float32) + btd_ref[...] + cmn
    ott_ref[...] = jnp.dot(ft, wtt_ref[...], preferred_element_type=jnp.float32) + btt_ref[...] + cmn


def _word_proj(feat_word, W_ww, W_wt, W_wd, b_ww, b_wt, b_wd):
    n = feat_word.shape[0]
    blk = pl.BlockSpec((_BR, D), lambda i: (i, 0))
    wblk = pl.BlockSpec((D, D), lambda i: (0, 0))
    bblk = pl.BlockSpec((1, D), lambda i: (0, 0))
    return pl.pallas_call(
        _word_proj_body,
        grid=(n // _BR,),
        in_specs=[blk, wblk, wblk, wblk, bblk, bblk, bblk],
        out_specs=[blk, blk, blk],
        out_shape=[jax.ShapeDtypeStruct((n, D), jnp.float32)] * 3,
    )(feat_word, W_ww, W_wt, W_wd, b_ww.reshape(1, D), b_wt.reshape(1, D),
      b_wd.reshape(1, D))


def _topic_proj(feat_topic, effect, W_td, W_tt, W_cau, W_noi, b_td, b_tt):
    n = feat_topic.shape[0]
    return pl.pallas_call(
        _topic_proj_body,
        out_shape=[jax.ShapeDtypeStruct((n, D), jnp.float32)] * 2,
    )(feat_topic, effect, W_td, W_tt, W_cau, W_noi, b_td.reshape(1, D),
      b_tt.reshape(1, D))


def _pad_edges(src, dst, w):
    e = src.shape[0]
    e_pad = _NSUB * _SB * (-(-e // (_NSUB * _SB)))
    pad = e_pad - e
    return (jnp.pad(src, (0, pad)),
            jnp.pad(dst, (0, pad), constant_values=-1),
            jnp.pad(w, (0, pad)))


def _iota16():
    return lax.iota(jnp.int32, 16)


def _splat16(x):
    return jnp.full((16,), x, jnp.int32)


def _sc_agg_body(
    wh_ww, wh_wt, wh_wd, wh_td, wh_tt,
    ww_src, ww_dst, ww_w, wt_src, wt_dst, wt_w, wd_src, wd_dst, wd_w,
    td_src, td_dst, td_w, tt_src, tt_dst, tt_w,
    out_w, out_t, out_d,
    feat_sh, cnt_sh,
    scan_src, scan_dst, scan_w,
    stage_src, stage_dloc, stage_w,
    dma_src0, dma_dloc0, dma_w0, dma_src1, dma_dloc1, dma_w1,
    rows0, rows1, cbuf, ones_v, zc_v,
    gsem0, gsem1,
):
    cid = lax.axis_index("c")
    sid = lax.axis_index("s")

    for j in range(8):
        ones_v[pl.ds(j * 16, 16)] = jnp.ones((16,), jnp.float32)
        zc_v[pl.ds(j * 16, 16)] = jnp.zeros((16,), jnp.float32)

    rows_per_tile = _C // _NSUB
    tile_row0 = sid * rows_per_tile
    bufs = ((dma_src0, dma_dloc0, dma_w0, rows0, gsem0),
            (dma_src1, dma_dloc1, dma_w1, rows1, gsem1))

    def _fill(dma_s, dma_d, dma_wt):
        for j in range(8):
            s2 = pl.ds(j * 16, 16)
            dma_s[s2] = stage_src[s2]
            dma_d[s2] = stage_dloc[s2]
            dma_wt[s2] = stage_w[s2]

    def _scale(dma_wt, rws):
        def _srow(r, _):
            wv = plsc.load_gather(dma_wt, [_splat16(r)])
            for j in range(8):
                sl = pl.ds(j * 16, 16)
                rws[r, sl] = rws[r, sl] * wv
            return 0
        lax.fori_loop(0, 128, _srow, 0)

    def _drain(wh_hbm, par):
        dma_s, dma_d, dma_wt, rws, sem = bufs[par]
        pltpu.make_async_copy(wh_hbm.at[dma_s], rws, sem).wait()
        _scale(dma_wt, rws)
        pltpu.sync_copy(rws, feat_sh.at[dma_d], add=True)
        pltpu.sync_copy(ones_v, cnt_sh.at[dma_d], add=True)

    def _flush_event(wh_hbm, nf):
        def _go(par):
            dma_s, dma_d, dma_wt, rws, sem = bufs[par]
            _fill(dma_s, dma_d, dma_wt)
            pltpu.async_copy(wh_hbm.at[dma_s], rws, sem)

            @pl.when(nf > 0)
            def _():
                _drain(wh_hbm, 1 - par)
            return 0
        lax.cond(nf % 2 == 0, lambda: _go(0), lambda: _go(1))
        stage_src[pl.ds(0, 16)] = stage_src[pl.ds(128, 16)]
        stage_dloc[pl.ds(0, 16)] = stage_dloc[pl.ds(128, 16)]
        stage_w[pl.ds(0, 16)] = stage_w[pl.ds(128, 16)]

    def _pass(wh_hbm, src_hbm, dst_hbm, w_hbm, e_pad, out_hbm, lo, accumulate):
        def _zrow(r, _):
            for j in range(8):
                rows0[r, pl.ds(j * 16, 16)] = jnp.zeros((16,), jnp.float32)
            return 0
        lax.fori_loop(0, 128, _zrow, 0)
        for b in range(rows_per_tile // 128):
            pltpu.sync_copy(rows0, feat_sh.at[pl.ds(tile_row0 + b * 128, 128)])
            pltpu.sync_copy(zc_v, cnt_sh.at[pl.ds(tile_row0 + b * 128, 128)])
        plsc.subcore_barrier()

        pt = e_pad // _NSUB
        e0 = sid * pt
        hi = lo + _C

        def _scan_block(bi, carry):
            pos0 = e0 + bi * _SB
            pltpu.sync_copy(dst_hbm.at[pl.ds(pos0, _SB)], scan_dst)
            pltpu.sync_copy(src_hbm.at[pl.ds(pos0, _SB)], scan_src)
            pltpu.sync_copy(w_hbm.at[pl.ds(pos0, _SB)], scan_w)

            def _step(i, carry):
                k, nf = carry
                sl = pl.ds(i * 16, 16)
                dstv = scan_dst[sl]
                inr = (dstv >= lo) & (dstv < hi)
                inr32 = jnp.where(inr, _splat16(1), _splat16(0))
                ps = plsc.cumsum(inr32)
                pos = jnp.where(inr, _splat16(k) + ps - 1, _splat16(_STG - 8))
                plsc.store_scatter(stage_dloc, [pos], dstv - lo)
                plsc.store_scatter(stage_src, [pos], scan_src[sl])
                plsc.store_scatter(stage_w, [pos], scan_w[sl])
                k = k + jnp.sum(inr32)

                def _full(args):
                    k, nf = args
                    _flush_event(wh_hbm, nf)
                    return (k - 128, nf + 1)

                return lax.cond(k >= 128, _full, lambda a: a, (k, nf))

            return lax.fori_loop(0, _SB // 16, _step, carry)

        k, nf = lax.fori_loop(0, pt // _SB, _scan_block, (0, 0))

        @pl.when(nf > 0)
        def _():
            lax.cond((nf - 1) % 2 == 0,
                     lambda: (_drain(wh_hbm, 0), 0)[1],
                     lambda: (_drain(wh_hbm, 1), 0)[1])
        for j in range(8):
            s2 = pl.ds(j * 16, 16)
            lanes = _splat16(j * 16) + _iota16()
            live = lanes < _splat16(k)
            dma_src0[s2] = jnp.where(live, stage_src[s2], _splat16(0))
            dma_dloc0[s2] = jnp.where(live, stage_dloc[s2], _splat16(_C))
            dma_w0[s2] = stage_w[s2]
        pltpu.sync_copy(wh_hbm.at[dma_src0], rows0)
        _scale(dma_w0, rows0)
        pltpu.sync_copy(rows0, feat_sh.at[dma_dloc0], add=True)
        pltpu.sync_copy(ones_v, cnt_sh.at[dma_dloc0], add=True)
        plsc.subcore_barrier()

        def _finblk(b, _):
            row0 = tile_row0 + b * 64
            pltpu.sync_copy(feat_sh.at[pl.ds(row0, 64)], rows0.at[pl.ds(0, 64)])
            pltpu.sync_copy(cnt_sh.at[pl.ds(row0, 64)], cbuf.at[pl.ds(0, 64)])
            if accumulate:
                pltpu.sync_copy(out_hbm.at[pl.ds(lo + row0, 64)],
                                rows0.at[pl.ds(64, 64)])

            def _fin(r, _):
                c = plsc.load_gather(cbuf, [_splat16(r)])
                recip = jnp.where(c > 0, 1.0 / jnp.maximum(c, 1.0), 0.0)
                for j in range(8):
                    sl = pl.ds(j * 16, 16)
                    if accumulate:
                        rows0[r, sl] = rows0[r, sl] * recip + rows0[r + 64, sl]
                    else:
                        rows0[r, sl] = rows0[r, sl] * recip
                return 0
            lax.fori_loop(0, 64, _fin, 0)
            pltpu.sync_copy(rows0.at[pl.ds(0, 64)],
                            out_hbm.at[pl.ds(lo + row0, 64)])
            return 0
        lax.fori_loop(0, rows_per_tile // 64, _finblk, 0)
        plsc.subcore_barrier()

    def _etype(wh_hbm, src_hbm, dst_hbm, w_hbm, e_pad, out_hbm, nch,
               accumulate, mode):
        if mode == "split":
            trips = -(-nch // 2)

            def _chunk(ci, _):
                chunk = 2 * ci + cid

                @pl.when(chunk < nch)
                def _():
                    _pass(wh_hbm, src_hbm, dst_hbm, w_hbm, e_pad, out_hbm,
                          chunk * _C, accumulate)
                return 0
            lax.fori_loop(0, trips, _chunk, 0)
        else:
            core = 0 if mode == "core0" else 1

            @pl.when(cid == core)
            def _():
                def _chunk(ci, _):
                    _pass(wh_hbm, src_hbm, dst_hbm, w_hbm, e_pad, out_hbm,
                          ci * _C, accumulate)
                    return 0
                lax.fori_loop(0, nch, _chunk, 0)

    _etype(wh_ww, ww_src, ww_dst, ww_w, ww_src.shape[0], out_w, _NCH_W,
           False, "split")
    _etype(wh_wt, wt_src, wt_dst, wt_w, wt_src.shape[0], out_t, _NCH_T,
           False, "core0")
    _etype(wh_tt, tt_src, tt_dst, tt_w, tt_src.shape[0], out_t, _NCH_T,
           True, "core0")
    _etype(wh_wd, wd_src, wd_dst, wd_w, wd_src.shape[0], out_d, _NCH_D,
           False, "core1")
    _etype(wh_td, td_src, td_dst, td_w, td_src.shape[0], out_d, _NCH_D,
           True, "core1")


def _sc_aggregate(wh_ww, wh_wt, wh_wd, wh_td, wh_tt, edges):
    (ww_src, ww_dst, ww_w, wt_src, wt_dst, wt_w, wd_src, wd_dst, wd_w,
     td_src, td_dst, td_w, tt_src, tt_dst, tt_w) = edges
    f32 = jnp.float32
    i32 = jnp.int32
    mesh = plsc.VectorSubcoreMesh(core_axis_name="c", subcore_axis_name="s")
    run = pl.kernel(
        _sc_agg_body,
        mesh=mesh,
        compiler_params=pltpu.CompilerParams(needs_layout_passes=False),
        out_type=[
            jax.ShapeDtypeStruct((_NWP, D), f32),
            jax.ShapeDtypeStruct((_NTP, D), f32),
            jax.ShapeDtypeStruct((_NDP, D), f32),
        ],
        scratch_types=[
            pltpu.VMEM_SHARED((_C + 8, D), f32),
            pltpu.VMEM_SHARED((_C + 8,), f32),
            pltpu.VMEM((_SB,), i32),
            pltpu.VMEM((_SB,), i32),
            pltpu.VMEM((_SB,), f32),
            pltpu.VMEM((_STG,), i32),
            pltpu.VMEM((_STG,), i32),
            pltpu.VMEM((_STG,), f32),
            pltpu.VMEM((128,), i32),
            pltpu.VMEM((128,), i32),
            pltpu.VMEM((128,), f32),
            pltpu.VMEM((128,), i32),
            pltpu.VMEM((128,), i32),
            pltpu.VMEM((128,), f32),
            pltpu.VMEM((128, D), f32),
            pltpu.VMEM((128, D), f32),
            pltpu.VMEM((128,), f32),
            pltpu.VMEM((128,), f32),
            pltpu.VMEM((128,), f32),
            pltpu.SemaphoreType.DMA,
            pltpu.SemaphoreType.DMA,
        ],
    )
    return run(wh_ww, wh_wt, wh_wd, wh_td, wh_tt,
               ww_src, ww_dst, ww_w, wt_src, wt_dst, wt_w,
               wd_src, wd_dst, wd_w, td_src, td_dst, td_w,
               tt_src, tt_dst, tt_w)


def kernel(feat_word, feat_topic, effect, ww_w, wt_w, wd_w, td_w, tt_w,
           W_ww, b_ww, W_wt, b_wt, W_wd, b_wd, W_td, b_td, W_tt, b_tt,
           W_cau, W_noi,
           ww_src, ww_dst, wt_src, wt_dst, wd_src, wd_dst,
           td_src, td_dst, tt_src, tt_dst):
    wh_ww, wh_wt, wh_wd = _word_proj(feat_word, W_ww, W_wt, W_wd,
                                     b_ww, b_wt, b_wd)
    wh_td, wh_tt = _topic_proj(feat_topic, effect, W_td, W_tt, W_cau, W_noi,
                               b_td, b_tt)
    flat = []
    for t in (_pad_edges(ww_src, ww_dst, ww_w),
              _pad_edges(wt_src, wt_dst, wt_w),
              _pad_edges(wd_src, wd_dst, wd_w),
              _pad_edges(td_src, td_dst, td_w),
              _pad_edges(tt_src, tt_dst, tt_w)):
        flat.extend(t)
    h_word, h_topic, h_doc = _sc_aggregate(wh_ww, wh_wt, wh_wd, wh_td, wh_tt,
                                           tuple(flat))
    return jnp.concatenate(
        [h_word[:NW], h_topic[:NT], h_doc[:ND]], axis=0)


# speedup vs baseline: 2.5610x; 1.1189x over previous
import functools

import jax
import jax.numpy as jnp
from jax import lax
from jax.experimental import pallas as pl
from jax.experimental.pallas import tpu as pltpu
from jax.experimental.pallas import tpu_sc as plsc

NW, NT, ND, D = 100000, 5000, 20000, 128
_BR = 2000
_C = 10240
_SB = 1024
_STG = 160
_NSUB = 16

_NCH_W = -(-NW // _C)
_NCH_T = -(-NT // _C)
_NCH_D = -(-ND // _C)
_NWP = _NCH_W * _C
_NTP = _NCH_T * _C
_NDP = _NCH_D * _C


def _word_proj_body(x_ref, www_ref, wwt_ref, wwd_ref, bww_ref, bwt_ref,
                    bwd_ref, o1_ref, o2_ref, o3_ref):
    x = x_ref[...]
    o1_ref[...] = jnp.dot(x, www_ref[...], preferred_element_type=jnp.float32) + bww_ref[...]
    o2_ref[...] = jnp.dot(x, wwt_ref[...], preferred_element_type=jnp.float32) + bwt_ref[...]
    o3_ref[...] = jnp.dot(x, wwd_ref[...], preferred_element_type=jnp.float32) + bwd_ref[...]


def _topic_proj_body(ft_ref, eff_ref, wtd_ref, wtt_ref, wcau_ref, wnoi_ref,
                     btd_ref, btt_ref, otd_ref, ott_ref):
    ft = ft_ref[...]
    eff = eff_ref[...]
    pos = (eff > 0).astype(jnp.float32)
    neg = (eff < 0).astype(jnp.float32)
    cau = jnp.dot(ft * pos, wcau_ref[...], preferred_element_type=jnp.float32)
    noi = jnp.dot(ft * neg, wnoi_ref[...], preferred_element_type=jnp.float32)
    cmn = cau - noi
    otd_ref[...] = jnp.dot(ft, wtd_ref[...], preferred_element_type=jnp.float32) + btd_ref[...] + cmn
    ott_ref[...] = jnp.dot(ft, wtt_ref[...], preferred_element_type=jnp.float32) + btt_ref[...] + cmn


def _word_proj(feat_word, W_ww, W_wt, W_wd, b_ww, b_wt, b_wd):
    n = feat_word.shape[0]
    blk = pl.BlockSpec((_BR, D), lambda i: (i, 0))
    wblk = pl.BlockSpec((D, D), lambda i: (0, 0))
    bblk = pl.BlockSpec((1, D), lambda i: (0, 0))
    return pl.pallas_call(
        _word_proj_body,
        grid=(n // _BR,),
        in_specs=[blk, wblk, wblk, wblk, bblk, bblk, bblk],
        out_specs=[blk, blk, blk],
        out_shape=[jax.ShapeDtypeStruct((n, D), jnp.float32)] * 3,
    )(feat_word, W_ww, W_wt, W_wd, b_ww.reshape(1, D), b_wt.reshape(1, D),
      b_wd.reshape(1, D))


def _topic_proj(feat_topic, effect, W_td, W_tt, W_cau, W_noi, b_td, b_tt):
    n = feat_topic.shape[0]
    return pl.pallas_call(
        _topic_proj_body,
        out_shape=[jax.ShapeDtypeStruct((n, D), jnp.float32)] * 2,
    )(feat_topic, effect, W_td, W_tt, W_cau, W_noi, b_td.reshape(1, D),
      b_tt.reshape(1, D))


def _pad_edges(src, dst, w):
    e = src.shape[0]
    unit = _NSUB * _SB * 2
    e_pad = unit * (-(-e // unit))
    pad = e_pad - e
    return (jnp.pad(src, (0, pad)),
            jnp.pad(dst, (0, pad), constant_values=-1),
            jnp.pad(w, (0, pad)))


def _iota16():
    return lax.iota(jnp.int32, 16)


def _splat16(x):
    return jnp.full((16,), x, jnp.int32)


def _sc_agg_body(
    wh_ww, wh_wt, wh_wd, wh_td, wh_tt,
    ww_src, ww_dst, ww_w, wt_src, wt_dst, wt_w, wd_src, wd_dst, wd_w,
    td_src, td_dst, td_w, tt_src, tt_dst, tt_w,
    out_w, out_t, out_d,
    feat_sh, cnt_sh,
    scan_src0, scan_dst0, scan_w0, scan_src1, scan_dst1, scan_w1,
    stage_src, stage_dloc, stage_w,
    dma_src0, dma_dloc0, dma_w0, dma_src1, dma_dloc1, dma_w1,
    rows0, rows1, cbuf, ones_v, zc_v,
    gsem0, gsem1, ssem0, ssem1, psem0, psem1,
):
    cid = lax.axis_index("c")
    sid = lax.axis_index("s")

    for j in range(8):
        ones_v[pl.ds(j * 16, 16)] = jnp.ones((16,), jnp.float32)
        zc_v[pl.ds(j * 16, 16)] = jnp.zeros((16,), jnp.float32)

    rows_per_tile = _C // _NSUB
    tile_row0 = sid * rows_per_tile
    bufs = ((dma_src0, dma_dloc0, dma_w0, rows0, gsem0, ssem0),
            (dma_src1, dma_dloc1, dma_w1, rows1, gsem1, ssem1))
    sbufs = ((scan_src0, scan_dst0, scan_w0, psem0),
             (scan_src1, scan_dst1, scan_w1, psem1))

    def _fill(dma_s, dma_d, dma_wt):
        for j in range(8):
            s2 = pl.ds(j * 16, 16)
            dma_s[s2] = stage_src[s2]
            dma_d[s2] = stage_dloc[s2]
            dma_wt[s2] = stage_w[s2]

    def _scale(dma_wt, rws):
        def _srow(r, _):
            wv = plsc.load_gather(dma_wt, [_splat16(r)])
            for j in range(8):
                sl = pl.ds(j * 16, 16)
                rws[r, sl] = rws[r, sl] * wv
            return 0
        lax.fori_loop(0, 128, _srow, 0)

    def _drain(wh_hbm, par):
        dma_s, dma_d, dma_wt, rws, gsem, ssem = bufs[par]
        pltpu.make_async_copy(wh_hbm.at[dma_s], rws, gsem).wait()
        _scale(dma_wt, rws)
        pltpu.async_copy(rws, feat_sh.at[dma_d], ssem, add=True)
        pltpu.async_copy(ones_v, cnt_sh.at[dma_d], ssem, add=True)

    def _wait_scatter(par):
        dma_s, dma_d, dma_wt, rws, gsem, ssem = bufs[par]
        pltpu.make_async_copy(rws, feat_sh.at[dma_d], ssem).wait()
        pltpu.make_async_copy(ones_v, cnt_sh.at[dma_d], ssem).wait()

    def _flush_event(wh_hbm, nf):
        def _go(par):
            dma_s, dma_d, dma_wt, rws, gsem, ssem = bufs[par]

            @pl.when(nf >= 2)
            def _():
                _wait_scatter(par)
            _fill(dma_s, dma_d, dma_wt)
            pltpu.async_copy(wh_hbm.at[dma_s], rws, gsem)

            @pl.when(nf > 0)
            def _():
                _drain(wh_hbm, 1 - par)
            return 0
        lax.cond(nf % 2 == 0, lambda: _go(0), lambda: _go(1))
        stage_src[pl.ds(0, 16)] = stage_src[pl.ds(128, 16)]
        stage_dloc[pl.ds(0, 16)] = stage_dloc[pl.ds(128, 16)]
        stage_w[pl.ds(0, 16)] = stage_w[pl.ds(128, 16)]

    def _pass(wh_hbm, src_hbm, dst_hbm, w_hbm, e_pad, out_hbm, lo, accumulate):
        def _zrow(r, _):
            for j in range(8):
                rows0[r, pl.ds(j * 16, 16)] = jnp.zeros((16,), jnp.float32)
            return 0
        lax.fori_loop(0, 128, _zrow, 0)
        for b in range(rows_per_tile // 128):
            pltpu.sync_copy(rows0, feat_sh.at[pl.ds(tile_row0 + b * 128, 128)])
            pltpu.sync_copy(zc_v, cnt_sh.at[pl.ds(tile_row0 + b * 128, 128)])
        plsc.subcore_barrier()

        pt = e_pad // _NSUB
        e0 = sid * pt
        hi = lo + _C

        def _prefetch(sb, pos0):
            ss, sd, sw, psem = sb
            pltpu.async_copy(dst_hbm.at[pl.ds(pos0, _SB)], sd, psem)
            pltpu.async_copy(src_hbm.at[pl.ds(pos0, _SB)], ss, psem)
            pltpu.async_copy(w_hbm.at[pl.ds(pos0, _SB)], sw, psem)

        def _wait_pf(sb):
            ss, sd, sw, psem = sb
            pltpu.make_async_copy(dst_hbm.at[pl.ds(0, _SB)], sd, psem).wait()
            pltpu.make_async_copy(src_hbm.at[pl.ds(0, _SB)], ss, psem).wait()
            pltpu.make_async_copy(w_hbm.at[pl.ds(0, _SB)], sw, psem).wait()

        def _steps(sb, carry):
            ss, sd, sw, _ = sb

            def _step(i, carry):
                k, nf = carry
                sl = pl.ds(i * 16, 16)
                dstv = sd[sl]
                inr = (dstv >= lo) & (dstv < hi)
                inr32 = jnp.where(inr, _splat16(1), _splat16(0))
                ps = plsc.cumsum(inr32)
                pos = jnp.where(inr, _splat16(k) + ps - 1, _splat16(_STG - 8))
                plsc.store_scatter(stage_dloc, [pos], dstv - lo)
                plsc.store_scatter(stage_src, [pos], ss[sl])
                plsc.store_scatter(stage_w, [pos], sw[sl])
                k = k + ps[15]

                def _full(args):
                    k, nf = args
                    _flush_event(wh_hbm, nf)
                    return (k - 128, nf + 1)

                return lax.cond(k >= 128, _full, lambda a: a, (k, nf))

            return lax.fori_loop(0, _SB // 16, _step, carry)

        def _pair(bp, carry):
            base = e0 + bp * 2 * _SB
            _wait_pf(sbufs[0])
            _prefetch(sbufs[1], base + _SB)
            carry = _steps(sbufs[0], carry)
            _wait_pf(sbufs[1])
            nxt = jnp.minimum(base + 2 * _SB, e0 + pt - _SB)
            _prefetch(sbufs[0], nxt)
            carry = _steps(sbufs[1], carry)
            return carry

        _prefetch(sbufs[0], e0)
        k, nf = lax.fori_loop(0, pt // (2 * _SB), _pair, (0, 0))
        _wait_pf(sbufs[0])

        @pl.when(nf > 0)
        def _():
            lax.cond((nf - 1) % 2 == 0,
                     lambda: (_drain(wh_hbm, 0), 0)[1],
                     lambda: (_drain(wh_hbm, 1), 0)[1])

        @pl.when(nf >= 2)
        def _():
            lax.cond(nf % 2 == 0,
                     lambda: (_wait_scatter(0), 0)[1],
                     lambda: (_wait_scatter(1), 0)[1])

        @pl.when(nf >= 1)
        def _():
            lax.cond((nf - 1) % 2 == 0,
                     lambda: (_wait_scatter(0), 0)[1],
                     lambda: (_wait_scatter(1), 0)[1])

        for j in range(8):
            s2 = pl.ds(j * 16, 16)
            lanes = _splat16(j * 16) + _iota16()
            live = lanes < _splat16(k)
            dma_src0[s2] = jnp.where(live, stage_src[s2], _splat16(0))
            dma_dloc0[s2] = jnp.where(live, stage_dloc[s2], _splat16(_C))
            dma_w0[s2] = stage_w[s2]
        pltpu.sync_copy(wh_hbm.at[dma_src0], rows0)
        _scale(dma_w0, rows0)
        pltpu.sync_copy(rows0, feat_sh.at[dma_dloc0], add=True)
        pltpu.sync_copy(ones_v, cnt_sh.at[dma_dloc0], add=True)
        plsc.subcore_barrier()

        def _finblk(b, _):
            row0 = tile_row0 + b * 64
            pltpu.sync_copy(feat_sh.at[pl.ds(row0, 64)], rows0.at[pl.ds(0, 64)])
            pltpu.sync_copy(cnt_sh.at[pl.ds(row0, 64)], cbuf.at[pl.ds(0, 64)])
            if accumulate:
                pltpu.sync_copy(out_hbm.at[pl.ds(lo + row0, 64)],
                                rows0.at[pl.ds(64, 64)])

            def _fin(r, _):
                c = plsc.load_gather(cbuf, [_splat16(r)])
                recip = jnp.where(c > 0, 1.0 / jnp.maximum(c, 1.0), 0.0)
                for j in range(8):
                    sl = pl.ds(j * 16, 16)
                    if accumulate:
                        rows0[r, sl] = rows0[r, sl] * recip + rows0[r + 64, sl]
                    else:
                        rows0[r, sl] = rows0[r, sl] * recip
                return 0
            lax.fori_loop(0, 64, _fin, 0)
            pltpu.sync_copy(rows0.at[pl.ds(0, 64)],
                            out_hbm.at[pl.ds(lo + row0, 64)])
            return 0
        lax.fori_loop(0, rows_per_tile // 64, _finblk, 0)
        plsc.subcore_barrier()

    def _etype(wh_hbm, src_hbm, dst_hbm, w_hbm, e_pad, out_hbm, nch,
               accumulate, mode):
        if mode == "split":
            trips = -(-nch // 2)

            def _chunk(ci, _):
                chunk = 2 * ci + cid

                @pl.when(chunk < nch)
                def _():
                    _pass(wh_hbm, src_hbm, dst_hbm, w_hbm, e_pad, out_hbm,
                          chunk * _C, accumulate)
                return 0
            lax.fori_loop(0, trips, _chunk, 0)
        else:
            core = 0 if mode == "core0" else 1

            @pl.when(cid == core)
            def _():
                def _chunk(ci, _):
                    _pass(wh_hbm, src_hbm, dst_hbm, w_hbm, e_pad, out_hbm,
                          ci * _C, accumulate)
                    return 0
                lax.fori_loop(0, nch, _chunk, 0)

    _etype(wh_ww, ww_src, ww_dst, ww_w, ww_src.shape[0], out_w, _NCH_W,
           False, "split")
    _etype(wh_wt, wt_src, wt_dst, wt_w, wt_src.shape[0], out_t, _NCH_T,
           False, "core0")
    _etype(wh_tt, tt_src, tt_dst, tt_w, tt_src.shape[0], out_t, _NCH_T,
           True, "core0")
    _etype(wh_wd, wd_src, wd_dst, wd_w, wd_src.shape[0], out_d, _NCH_D,
           False, "core1")
    _etype(wh_td, td_src, td_dst, td_w, td_src.shape[0], out_d, _NCH_D,
           True, "core1")


def _sc_aggregate(wh_ww, wh_wt, wh_wd, wh_td, wh_tt, edges):
    (ww_src, ww_dst, ww_w, wt_src, wt_dst, wt_w, wd_src, wd_dst, wd_w,
     td_src, td_dst, td_w, tt_src, tt_dst, tt_w) = edges
    f32 = jnp.float32
    i32 = jnp.int32
    mesh = plsc.VectorSubcoreMesh(core_axis_name="c", subcore_axis_name="s")
    run = pl.kernel(
        _sc_agg_body,
        mesh=mesh,
        compiler_params=pltpu.CompilerParams(needs_layout_passes=False),
        out_type=[
            jax.ShapeDtypeStruct((_NWP, D), f32),
            jax.ShapeDtypeStruct((_NTP, D), f32),
            jax.ShapeDtypeStruct((_NDP, D), f32),
        ],
        scratch_types=[
            pltpu.VMEM_SHARED((_C + 8, D), f32),
            pltpu.VMEM_SHARED((_C + 8,), f32),
            pltpu.VMEM((_SB,), i32),
            pltpu.VMEM((_SB,), i32),
            pltpu.VMEM((_SB,), f32),
            pltpu.VMEM((_SB,), i32),
            pltpu.VMEM((_SB,), i32),
            pltpu.VMEM((_SB,), f32),
            pltpu.VMEM((_STG,), i32),
            pltpu.VMEM((_STG,), i32),
            pltpu.VMEM((_STG,), f32),
            pltpu.VMEM((128,), i32),
            pltpu.VMEM((128,), i32),
            pltpu.VMEM((128,), f32),
            pltpu.VMEM((128,), i32),
            pltpu.VMEM((128,), i32),
            pltpu.VMEM((128,), f32),
            pltpu.VMEM((128, D), f32),
            pltpu.VMEM((128, D), f32),
            pltpu.VMEM((128,), f32),
            pltpu.VMEM((128,), f32),
            pltpu.VMEM((128,), f32),
            pltpu.SemaphoreType.DMA,
            pltpu.SemaphoreType.DMA,
            pltpu.SemaphoreType.DMA,
            pltpu.SemaphoreType.DMA,
            pltpu.SemaphoreType.DMA,
            pltpu.SemaphoreType.DMA,
        ],
    )
    return run(wh_ww, wh_wt, wh_wd, wh_td, wh_tt,
               ww_src, ww_dst, ww_w, wt_src, wt_dst, wt_w,
               wd_src, wd_dst, wd_w, td_src, td_dst, td_w,
               tt_src, tt_dst, tt_w)


def kernel(feat_word, feat_topic, effect, ww_w, wt_w, wd_w, td_w, tt_w,
           W_ww, b_ww, W_wt, b_wt, W_wd, b_wd, W_td, b_td, W_tt, b_tt,
           W_cau, W_noi,
           ww_src, ww_dst, wt_src, wt_dst, wd_src, wd_dst,
           td_src, td_dst, tt_src, tt_dst):
    wh_ww, wh_wt, wh_wd = _word_proj(feat_word, W_ww, W_wt, W_wd,
                                     b_ww, b_wt, b_wd)
    wh_td, wh_tt = _topic_proj(feat_topic, effect, W_td, W_tt, W_cau, W_noi,
                               b_td, b_tt)
    flat = []
    for t in (_pad_edges(ww_src, ww_dst, ww_w),
              _pad_edges(wt_src, wt_dst, wt_w),
              _pad_edges(wd_src, wd_dst, wd_w),
              _pad_edges(td_src, td_dst, td_w),
              _pad_edges(tt_src, tt_dst, tt_w)):
        flat.extend(t)
    h_word, h_topic, h_doc = _sc_aggregate(wh_ww, wh_wt, wh_wd, wh_td, wh_tt,
                                           tuple(flat))
    return jnp.concatenate(
        [h_word[:NW], h_topic[:NT], h_doc[:ND]], axis=0)


# speedup vs baseline: 2.8065x; 1.0959x over previous
import functools

import jax
import jax.numpy as jnp
from jax import lax
from jax.experimental import pallas as pl
from jax.experimental.pallas import tpu as pltpu
from jax.experimental.pallas import tpu_sc as plsc

NW, NT, ND, D = 100000, 5000, 20000, 128
_BR = 2000
_C = 10240
_SB = 1024
_STG = 160
_NSUB = 16

_NCH_W = -(-NW // _C)
_NCH_T = -(-NT // _C)
_NCH_D = -(-ND // _C)
_NWP = _NCH_W * _C
_NTP = _NCH_T * _C
_NDP = _NCH_D * _C


def _word_proj_body(x_ref, www_ref, wwt_ref, wwd_ref, bww_ref, bwt_ref,
                    bwd_ref, o1_ref, o2_ref, o3_ref):
    x = x_ref[...]
    o1_ref[...] = jnp.dot(x, www_ref[...], preferred_element_type=jnp.float32) + bww_ref[...]
    o2_ref[...] = jnp.dot(x, wwt_ref[...], preferred_element_type=jnp.float32) + bwt_ref[...]
    o3_ref[...] = jnp.dot(x, wwd_ref[...], preferred_element_type=jnp.float32) + bwd_ref[...]


def _topic_proj_body(ft_ref, eff_ref, wtd_ref, wtt_ref, wcau_ref, wnoi_ref,
                     btd_ref, btt_ref, otd_ref, ott_ref):
    ft = ft_ref[...]
    eff = eff_ref[...]
    pos = (eff > 0).astype(jnp.float32)
    neg = (eff < 0).astype(jnp.float32)
    cau = jnp.dot(ft * pos, wcau_ref[...], preferred_element_type=jnp.float32)
    noi = jnp.dot(ft * neg, wnoi_ref[...], preferred_element_type=jnp.float32)
    cmn = cau - noi
    otd_ref[...] = jnp.dot(ft, wtd_ref[...], preferred_element_type=jnp.float32) + btd_ref[...] + cmn
    ott_ref[...] = jnp.dot(ft, wtt_ref[...], preferred_element_type=jnp.float32) + btt_ref[...] + cmn


def _word_proj(feat_word, W_ww, W_wt, W_wd, b_ww, b_wt, b_wd):
    n = feat_word.shape[0]
    blk = pl.BlockSpec((_BR, D), lambda i: (i, 0))
    wblk = pl.BlockSpec((D, D), lambda i: (0, 0))
    bblk = pl.BlockSpec((1, D), lambda i: (0, 0))
    return pl.pallas_call(
        _word_proj_body,
        grid=(n // _BR,),
        in_specs=[blk, wblk, wblk, wblk, bblk, bblk, bblk],
        out_specs=[blk, blk, blk],
        out_shape=[jax.ShapeDtypeStruct((n, D), jnp.float32)] * 3,
    )(feat_word, W_ww, W_wt, W_wd, b_ww.reshape(1, D), b_wt.reshape(1, D),
      b_wd.reshape(1, D))


def _topic_proj(feat_topic, effect, W_td, W_tt, W_cau, W_noi, b_td, b_tt):
    n = feat_topic.shape[0]
    return pl.pallas_call(
        _topic_proj_body,
        out_shape=[jax.ShapeDtypeStruct((n, D), jnp.float32)] * 2,
    )(feat_topic, effect, W_td, W_tt, W_cau, W_noi, b_td.reshape(1, D),
      b_tt.reshape(1, D))


def _pad_edges(src, dst, w):
    e = src.shape[0]
    unit = _NSUB * _SB * 2
    e_pad = unit * (-(-e // unit))
    pad = e_pad - e
    return (jnp.pad(src, (0, pad)),
            jnp.pad(dst, (0, pad), constant_values=-1),
            jnp.pad(w, (0, pad)))


def _iota16():
    return lax.iota(jnp.int32, 16)


def _splat16(x):
    return jnp.full((16,), x, jnp.int32)


def _sc_agg_body(
    wh_ww, wh_wt, wh_wd, wh_td, wh_tt,
    ww_src, ww_dst, ww_w, wt_src, wt_dst, wt_w, wd_src, wd_dst, wd_w,
    td_src, td_dst, td_w, tt_src, tt_dst, tt_w,
    out_hbm,
    feat_sh, cnt_sh,
    scan_src0, scan_dst0, scan_w0, scan_src1, scan_dst1, scan_w1,
    stage_src, stage_dloc, stage_w,
    dma_src0, dma_dloc0, dma_w0, dma_src1, dma_dloc1, dma_w1,
    rows0, rows1, cbuf, ones_v, zc_v,
    gsem0, gsem1, ssem0, ssem1, psem0, psem1,
):
    cid = lax.axis_index("c")
    sid = lax.axis_index("s")

    for j in range(8):
        ones_v[pl.ds(j * 16, 16)] = jnp.ones((16,), jnp.float32)
        zc_v[pl.ds(j * 16, 16)] = jnp.zeros((16,), jnp.float32)

    rows_per_tile = _C // _NSUB
    tile_row0 = sid * rows_per_tile
    bufs = ((dma_src0, dma_dloc0, dma_w0, rows0, gsem0, ssem0),
            (dma_src1, dma_dloc1, dma_w1, rows1, gsem1, ssem1))
    sbufs = ((scan_src0, scan_dst0, scan_w0, psem0),
             (scan_src1, scan_dst1, scan_w1, psem1))

    def _fill(dma_s, dma_d, dma_wt):
        for j in range(8):
            s2 = pl.ds(j * 16, 16)
            dma_s[s2] = stage_src[s2]
            dma_d[s2] = stage_dloc[s2]
            dma_wt[s2] = stage_w[s2]

    def _scale(dma_wt, rws):
        def _srow(r2, _):
            r = r2 * 2
            wv0 = plsc.load_gather(dma_wt, [_splat16(r)])
            wv1 = plsc.load_gather(dma_wt, [_splat16(r + 1)])
            for j in range(8):
                sl = pl.ds(j * 16, 16)
                rws[r, sl] = rws[r, sl] * wv0
            for j in range(8):
                sl = pl.ds(j * 16, 16)
                rws[r + 1, sl] = rws[r + 1, sl] * wv1
            return 0
        lax.fori_loop(0, 64, _srow, 0)

    def _drain(wh_hbm, par):
        dma_s, dma_d, dma_wt, rws, gsem, ssem = bufs[par]
        pltpu.make_async_copy(wh_hbm.at[dma_s], rws, gsem).wait()
        _scale(dma_wt, rws)
        pltpu.async_copy(rws, feat_sh.at[dma_d], ssem, add=True)
        pltpu.async_copy(ones_v, cnt_sh.at[dma_d], ssem, add=True)

    def _wait_scatter(par):
        dma_s, dma_d, dma_wt, rws, gsem, ssem = bufs[par]
        pltpu.make_async_copy(rws, feat_sh.at[dma_d], ssem).wait()
        pltpu.make_async_copy(ones_v, cnt_sh.at[dma_d], ssem).wait()

    def _flush_event(wh_hbm, nf):
        def _go(par):
            dma_s, dma_d, dma_wt, rws, gsem, ssem = bufs[par]

            @pl.when(nf >= 2)
            def _():
                _wait_scatter(par)
            _fill(dma_s, dma_d, dma_wt)
            pltpu.async_copy(wh_hbm.at[dma_s], rws, gsem)

            @pl.when(nf > 0)
            def _():
                _drain(wh_hbm, 1 - par)
            return 0
        lax.cond(nf % 2 == 0, lambda: _go(0), lambda: _go(1))
        stage_src[pl.ds(0, 16)] = stage_src[pl.ds(128, 16)]
        stage_dloc[pl.ds(0, 16)] = stage_dloc[pl.ds(128, 16)]
        stage_w[pl.ds(0, 16)] = stage_w[pl.ds(128, 16)]

    def _pass(wh_hbm, src_hbm, dst_hbm, w_hbm, e_pad, lo, accumulate,
              sec_base, n_valid):
        def _zrow(r, _):
            for j in range(8):
                rows0[r, pl.ds(j * 16, 16)] = jnp.zeros((16,), jnp.float32)
            return 0
        lax.fori_loop(0, 128, _zrow, 0)
        for b in range(rows_per_tile // 128):
            pltpu.sync_copy(rows0, feat_sh.at[pl.ds(tile_row0 + b * 128, 128)])
            pltpu.sync_copy(zc_v, cnt_sh.at[pl.ds(tile_row0 + b * 128, 128)])
        plsc.subcore_barrier()

        pt = e_pad // _NSUB
        e0 = sid * pt
        hi = lo + _C

        def _prefetch(sb, pos0):
            ss, sd, sw, psem = sb
            pltpu.async_copy(dst_hbm.at[pl.ds(pos0, _SB)], sd, psem)
            pltpu.async_copy(src_hbm.at[pl.ds(pos0, _SB)], ss, psem)
            pltpu.async_copy(w_hbm.at[pl.ds(pos0, _SB)], sw, psem)

        def _wait_pf(sb):
            ss, sd, sw, psem = sb
            pltpu.make_async_copy(dst_hbm.at[pl.ds(0, _SB)], sd, psem).wait()
            pltpu.make_async_copy(src_hbm.at[pl.ds(0, _SB)], ss, psem).wait()
            pltpu.make_async_copy(w_hbm.at[pl.ds(0, _SB)], sw, psem).wait()

        def _steps(sb, carry):
            ss, sd, sw, _ = sb

            def _step(i, carry):
                k, nf = carry
                sl = pl.ds(i * 16, 16)
                dstv = sd[sl]
                inr = (dstv >= lo) & (dstv < hi)
                inr32 = jnp.where(inr, _splat16(1), _splat16(0))
                ps = plsc.cumsum(inr32)
                pos = jnp.where(inr, _splat16(k) + ps - 1, _splat16(_STG - 8))
                plsc.store_scatter(stage_dloc, [pos], dstv - lo)
                plsc.store_scatter(stage_src, [pos], ss[sl])
                plsc.store_scatter(stage_w, [pos], sw[sl])
                k = k + ps[15]

                def _full(args):
                    k, nf = args
                    _flush_event(wh_hbm, nf)
                    return (k - 128, nf + 1)

                return lax.cond(k >= 128, _full, lambda a: a, (k, nf))

            return lax.fori_loop(0, _SB // 16, _step, carry)

        def _pair(bp, carry):
            base = e0 + bp * 2 * _SB
            _wait_pf(sbufs[0])
            _prefetch(sbufs[1], base + _SB)
            carry = _steps(sbufs[0], carry)
            _wait_pf(sbufs[1])
            nxt = jnp.minimum(base + 2 * _SB, e0 + pt - _SB)
            _prefetch(sbufs[0], nxt)
            carry = _steps(sbufs[1], carry)
            return carry

        _prefetch(sbufs[0], e0)
        k, nf = lax.fori_loop(0, pt // (2 * _SB), _pair, (0, 0))
        _wait_pf(sbufs[0])

        @pl.when(nf > 0)
        def _():
            lax.cond((nf - 1) % 2 == 0,
                     lambda: (_drain(wh_hbm, 0), 0)[1],
                     lambda: (_drain(wh_hbm, 1), 0)[1])

        @pl.when(nf >= 2)
        def _():
            lax.cond(nf % 2 == 0,
                     lambda: (_wait_scatter(0), 0)[1],
                     lambda: (_wait_scatter(1), 0)[1])

        @pl.when(nf >= 1)
        def _():
            lax.cond((nf - 1) % 2 == 0,
                     lambda: (_wait_scatter(0), 0)[1],
                     lambda: (_wait_scatter(1), 0)[1])

        for j in range(8):
            s2 = pl.ds(j * 16, 16)
            lanes = _splat16(j * 16) + _iota16()
            live = lanes < _splat16(k)
            dma_src0[s2] = jnp.where(live, stage_src[s2], _splat16(0))
            dma_dloc0[s2] = jnp.where(live, stage_dloc[s2], _splat16(_C))
            dma_w0[s2] = stage_w[s2]
        pltpu.sync_copy(wh_hbm.at[dma_src0], rows0)
        _scale(dma_w0, rows0)
        pltpu.sync_copy(rows0, feat_sh.at[dma_dloc0], add=True)
        pltpu.sync_copy(ones_v, cnt_sh.at[dma_dloc0], add=True)
        plsc.subcore_barrier()

        def _finblk(b, _):
            row0 = tile_row0 + b * 128
            gbase = sec_base + lo + row0
            local0 = lo + row0
            full = local0 + 128 <= n_valid
            pltpu.sync_copy(feat_sh.at[pl.ds(row0, 128)], rows0)
            pltpu.sync_copy(cnt_sh.at[pl.ds(row0, 128)], cbuf)
            if accumulate:
                def _rd_full():
                    pltpu.sync_copy(out_hbm.at[pl.ds(gbase, 128)], rows1)
                    return 0

                def _rd_part():
                    for q in range(16):
                        @pl.when(local0 + q * 8 < n_valid)
                        def _():
                            pltpu.sync_copy(
                                out_hbm.at[pl.ds(gbase + q * 8, 8)],
                                rows1.at[pl.ds(q * 8, 8)])
                    return 0
                lax.cond(full, _rd_full, _rd_part)

            def _fin(r, _):
                c = plsc.load_gather(cbuf, [_splat16(r)])
                recip = jnp.where(c > 0, 1.0 / jnp.maximum(c, 1.0), 0.0)
                for j in range(8):
                    sl = pl.ds(j * 16, 16)
                    if accumulate:
                        rows0[r, sl] = rows0[r, sl] * recip + rows1[r, sl]
                    else:
                        rows0[r, sl] = rows0[r, sl] * recip
                return 0
            lax.fori_loop(0, 128, _fin, 0)

            def _wr_full():
                pltpu.sync_copy(rows0, out_hbm.at[pl.ds(gbase, 128)])
                return 0

            def _wr_part():
                for q in range(16):
                    @pl.when(local0 + q * 8 < n_valid)
                    def _():
                        pltpu.sync_copy(rows0.at[pl.ds(q * 8, 8)],
                                        out_hbm.at[pl.ds(gbase + q * 8, 8)])
                return 0
            lax.cond(full, _wr_full, _wr_part)
            return 0
        lax.fori_loop(0, rows_per_tile // 128, _finblk, 0)
        plsc.subcore_barrier()

    def _etype(wh_hbm, src_hbm, dst_hbm, w_hbm, nch, accumulate, mode,
               sec_base, n_valid):
        e_pad = src_hbm.shape[0]
        if mode == "split":
            trips = -(-nch // 2)

            def _chunk(ci, _):
                chunk = 2 * ci + cid

                @pl.when(chunk < nch)
                def _():
                    _pass(wh_hbm, src_hbm, dst_hbm, w_hbm, e_pad,
                          chunk * _C, accumulate, sec_base, n_valid)
                return 0
            lax.fori_loop(0, trips, _chunk, 0)
        else:
            core = 0 if mode == "core0" else 1

            @pl.when(cid == core)
            def _():
                def _chunk(ci, _):
                    _pass(wh_hbm, src_hbm, dst_hbm, w_hbm, e_pad,
                          ci * _C, accumulate, sec_base, n_valid)
                    return 0
                lax.fori_loop(0, nch, _chunk, 0)

    _etype(wh_ww, ww_src, ww_dst, ww_w, _NCH_W, False, "split", 0, NW)
    _etype(wh_wt, wt_src, wt_dst, wt_w, _NCH_T, False, "core0", NW, NT)
    _etype(wh_tt, tt_src, tt_dst, tt_w, _NCH_T, True, "core0", NW, NT)
    _etype(wh_wd, wd_src, wd_dst, wd_w, _NCH_D, False, "core1", NW + NT, ND)
    _etype(wh_td, td_src, td_dst, td_w, _NCH_D, True, "core1", NW + NT, ND)


def _sc_aggregate(wh_ww, wh_wt, wh_wd, wh_td, wh_tt, edges):
    (ww_src, ww_dst, ww_w, wt_src, wt_dst, wt_w, wd_src, wd_dst, wd_w,
     td_src, td_dst, td_w, tt_src, tt_dst, tt_w) = edges
    f32 = jnp.float32
    i32 = jnp.int32
    mesh = plsc.VectorSubcoreMesh(core_axis_name="c", subcore_axis_name="s")
    run = pl.kernel(
        _sc_agg_body,
        mesh=mesh,
        compiler_params=pltpu.CompilerParams(needs_layout_passes=False),
        out_type=[
            jax.ShapeDtypeStruct((NW + NT + ND, D), f32),
        ],
        scratch_types=[
            pltpu.VMEM_SHARED((_C + 8, D), f32),
            pltpu.VMEM_SHARED((_C + 8,), f32),
            pltpu.VMEM((_SB,), i32),
            pltpu.VMEM((_SB,), i32),
            pltpu.VMEM((_SB,), f32),
            pltpu.VMEM((_SB,), i32),
            pltpu.VMEM((_SB,), i32),
            pltpu.VMEM((_SB,), f32),
            pltpu.VMEM((_STG,), i32),
            pltpu.VMEM((_STG,), i32),
            pltpu.VMEM((_STG,), f32),
            pltpu.VMEM((128,), i32),
            pltpu.VMEM((128,), i32),
            pltpu.VMEM((128,), f32),
            pltpu.VMEM((128,), i32),
            pltpu.VMEM((128,), i32),
            pltpu.VMEM((128,), f32),
            pltpu.VMEM((128, D), f32),
            pltpu.VMEM((128, D), f32),
            pltpu.VMEM((128,), f32),
            pltpu.VMEM((128,), f32),
            pltpu.VMEM((128,), f32),
            pltpu.SemaphoreType.DMA,
            pltpu.SemaphoreType.DMA,
            pltpu.SemaphoreType.DMA,
            pltpu.SemaphoreType.DMA,
            pltpu.SemaphoreType.DMA,
            pltpu.SemaphoreType.DMA,
        ],
    )
    return run(wh_ww, wh_wt, wh_wd, wh_td, wh_tt,
               ww_src, ww_dst, ww_w, wt_src, wt_dst, wt_w,
               wd_src, wd_dst, wd_w, td_src, td_dst, td_w,
               tt_src, tt_dst, tt_w)


def kernel(feat_word, feat_topic, effect, ww_w, wt_w, wd_w, td_w, tt_w,
           W_ww, b_ww, W_wt, b_wt, W_wd, b_wd, W_td, b_td, W_tt, b_tt,
           W_cau, W_noi,
           ww_src, ww_dst, wt_src, wt_dst, wd_src, wd_dst,
           td_src, td_dst, tt_src, tt_dst):
    wh_ww, wh_wt, wh_wd = _word_proj(feat_word, W_ww, W_wt, W_wd,
                                     b_ww, b_wt, b_wd)
    wh_td, wh_tt = _topic_proj(feat_topic, effect, W_td, W_tt, W_cau, W_noi,
                               b_td, b_tt)
    flat = []
    for t in (_pad_edges(ww_src, ww_dst, ww_w),
              _pad_edges(wt_src, wt_dst, wt_w),
              _pad_edges(wd_src, wd_dst, wd_w),
              _pad_edges(td_src, td_dst, td_w),
              _pad_edges(tt_src, tt_dst, tt_w)):
        flat.extend(t)
    (out,) = _sc_aggregate(wh_ww, wh_wt, wh_wd, wh_td, wh_tt, tuple(flat))
    return out


# speedup vs baseline: 2.9314x; 1.0445x over previous
import functools

import jax
import jax.numpy as jnp
from jax import lax
from jax.experimental import pallas as pl
from jax.experimental.pallas import tpu as pltpu
from jax.experimental.pallas import tpu_sc as plsc

NW, NT, ND, D = 100000, 5000, 20000, 128
_BR = 2000
_C = 10240
_SB = 1024
_STG = 192
_TRASH = 184
_NSUB = 16

_NCH_W = -(-NW // _C)
_NCH_T = -(-NT // _C)
_NCH_D = -(-ND // _C)
_NWP = _NCH_W * _C
_NTP = _NCH_T * _C
_NDP = _NCH_D * _C


def _word_proj_body(x_ref, www_ref, wwt_ref, wwd_ref, bww_ref, bwt_ref,
                    bwd_ref, o1_ref, o2_ref, o3_ref):
    x = x_ref[...]
    o1_ref[...] = jnp.dot(x, www_ref[...], preferred_element_type=jnp.float32) + bww_ref[...]
    o2_ref[...] = jnp.dot(x, wwt_ref[...], preferred_element_type=jnp.float32) + bwt_ref[...]
    o3_ref[...] = jnp.dot(x, wwd_ref[...], preferred_element_type=jnp.float32) + bwd_ref[...]


def _topic_proj_body(ft_ref, eff_ref, wtd_ref, wtt_ref, wcau_ref, wnoi_ref,
                     btd_ref, btt_ref, otd_ref, ott_ref):
    ft = ft_ref[...]
    eff = eff_ref[...]
    pos = (eff > 0).astype(jnp.float32)
    neg = (eff < 0).astype(jnp.float32)
    cau = jnp.dot(ft * pos, wcau_ref[...], preferred_element_type=jnp.float32)
    noi = jnp.dot(ft * neg, wnoi_ref[...], preferred_element_type=jnp.float32)
    cmn = cau - noi
    otd_ref[...] = jnp.dot(ft, wtd_ref[...], preferred_element_type=jnp.float32) + btd_ref[...] + cmn
    ott_ref[...] = jnp.dot(ft, wtt_ref[...], preferred_element_type=jnp.float32) + btt_ref[...] + cmn


def _word_proj(feat_word, W_ww, W_wt, W_wd, b_ww, b_wt, b_wd):
    n = feat_word.shape[0]
    blk = pl.BlockSpec((_BR, D), lambda i: (i, 0))
    wblk = pl.BlockSpec((D, D), lambda i: (0, 0))
    bblk = pl.BlockSpec((1, D), lambda i: (0, 0))
    return pl.pallas_call(
        _word_proj_body,
        grid=(n // _BR,),
        in_specs=[blk, wblk, wblk, wblk, bblk, bblk, bblk],
        out_specs=[blk, blk, blk],
        out_shape=[jax.ShapeDtypeStruct((n, D), jnp.float32)] * 3,
    )(feat_word, W_ww, W_wt, W_wd, b_ww.reshape(1, D), b_wt.reshape(1, D),
      b_wd.reshape(1, D))


def _topic_proj(feat_topic, effect, W_td, W_tt, W_cau, W_noi, b_td, b_tt):
    n = feat_topic.shape[0]
    return pl.pallas_call(
        _topic_proj_body,
        out_shape=[jax.ShapeDtypeStruct((n, D), jnp.float32)] * 2,
    )(feat_topic, effect, W_td, W_tt, W_cau, W_noi, b_td.reshape(1, D),
      b_tt.reshape(1, D))


def _pad_edges(src, dst, w):
    e = src.shape[0]
    unit = _NSUB * _SB * 2
    e_pad = unit * (-(-e // unit))
    pad = e_pad - e
    return (jnp.pad(src, (0, pad)),
            jnp.pad(dst, (0, pad), constant_values=-1),
            jnp.pad(w, (0, pad)))


def _iota16():
    return lax.iota(jnp.int32, 16)


def _splat16(x):
    return jnp.full((16,), x, jnp.int32)


def _sc_agg_body(
    wh_ww, wh_wt, wh_wd, wh_td, wh_tt,
    ww_src, ww_dst, ww_w, wt_src, wt_dst, wt_w, wd_src, wd_dst, wd_w,
    td_src, td_dst, td_w, tt_src, tt_dst, tt_w,
    out_hbm,
    feat_sh, cnt_sh,
    scan_src0, scan_dst0, scan_w0, scan_src1, scan_dst1, scan_w1,
    stage_src, stage_dloc, stage_w,
    dma_src0, dma_dloc0, dma_w0, dma_src1, dma_dloc1, dma_w1,
    rows0, rows1, cbuf, ones_v, zc_v,
    gsem0, gsem1, ssem0, ssem1, psem0, psem1,
):
    cid = lax.axis_index("c")
    sid = lax.axis_index("s")

    for j in range(8):
        ones_v[pl.ds(j * 16, 16)] = jnp.ones((16,), jnp.float32)
        zc_v[pl.ds(j * 16, 16)] = jnp.zeros((16,), jnp.float32)

    rows_per_tile = _C // _NSUB
    tile_row0 = sid * rows_per_tile
    bufs = ((dma_src0, dma_dloc0, dma_w0, rows0, gsem0, ssem0),
            (dma_src1, dma_dloc1, dma_w1, rows1, gsem1, ssem1))
    sbufs = ((scan_src0, scan_dst0, scan_w0, psem0),
             (scan_src1, scan_dst1, scan_w1, psem1))

    def _fill(dma_s, dma_d, dma_wt):
        for j in range(8):
            s2 = pl.ds(j * 16, 16)
            dma_s[s2] = stage_src[s2]
            dma_d[s2] = stage_dloc[s2]
            dma_wt[s2] = stage_w[s2]

    def _scale(dma_wt, rws):
        def _srow(r2, _):
            r = r2 * 2
            wv0 = plsc.load_gather(dma_wt, [_splat16(r)])
            wv1 = plsc.load_gather(dma_wt, [_splat16(r + 1)])
            for j in range(8):
                sl = pl.ds(j * 16, 16)
                rws[r, sl] = rws[r, sl] * wv0
            for j in range(8):
                sl = pl.ds(j * 16, 16)
                rws[r + 1, sl] = rws[r + 1, sl] * wv1
            return 0
        lax.fori_loop(0, 64, _srow, 0)

    def _drain(wh_hbm, par):
        dma_s, dma_d, dma_wt, rws, gsem, ssem = bufs[par]
        pltpu.make_async_copy(wh_hbm.at[dma_s], rws, gsem).wait()
        _scale(dma_wt, rws)
        pltpu.async_copy(rws, feat_sh.at[dma_d], ssem, add=True)
        pltpu.async_copy(ones_v, cnt_sh.at[dma_d], ssem, add=True)

    def _wait_scatter(par):
        dma_s, dma_d, dma_wt, rws, gsem, ssem = bufs[par]
        pltpu.make_async_copy(rws, feat_sh.at[dma_d], ssem).wait()
        pltpu.make_async_copy(ones_v, cnt_sh.at[dma_d], ssem).wait()

    def _flush_event(wh_hbm, nf):
        def _go(par):
            dma_s, dma_d, dma_wt, rws, gsem, ssem = bufs[par]

            @pl.when(nf >= 2)
            def _():
                _wait_scatter(par)
            _fill(dma_s, dma_d, dma_wt)
            pltpu.async_copy(wh_hbm.at[dma_s], rws, gsem)

            @pl.when(nf > 0)
            def _():
                _drain(wh_hbm, 1 - par)
            return 0
        lax.cond(nf % 2 == 0, lambda: _go(0), lambda: _go(1))
        for t in range(2):
            stage_src[pl.ds(t * 16, 16)] = stage_src[pl.ds(128 + t * 16, 16)]
            stage_dloc[pl.ds(t * 16, 16)] = stage_dloc[pl.ds(128 + t * 16, 16)]
            stage_w[pl.ds(t * 16, 16)] = stage_w[pl.ds(128 + t * 16, 16)]

    def _pass(wh_hbm, src_hbm, dst_hbm, w_hbm, e_pad, lo, accumulate,
              sec_base, n_valid):
        def _zrow(r, _):
            for j in range(8):
                rows0[r, pl.ds(j * 16, 16)] = jnp.zeros((16,), jnp.float32)
            return 0
        lax.fori_loop(0, 128, _zrow, 0)
        for b in range(rows_per_tile // 128):
            pltpu.sync_copy(rows0, feat_sh.at[pl.ds(tile_row0 + b * 128, 128)])
            pltpu.sync_copy(zc_v, cnt_sh.at[pl.ds(tile_row0 + b * 128, 128)])
        plsc.subcore_barrier()

        pt = e_pad // _NSUB
        e0 = sid * pt
        hi = lo + _C

        def _prefetch(sb, pos0):
            ss, sd, sw, psem = sb
            pltpu.async_copy(dst_hbm.at[pl.ds(pos0, _SB)], sd, psem)
            pltpu.async_copy(src_hbm.at[pl.ds(pos0, _SB)], ss, psem)
            pltpu.async_copy(w_hbm.at[pl.ds(pos0, _SB)], sw, psem)

        def _wait_pf(sb):
            ss, sd, sw, psem = sb
            pltpu.make_async_copy(dst_hbm.at[pl.ds(0, _SB)], sd, psem).wait()
            pltpu.make_async_copy(src_hbm.at[pl.ds(0, _SB)], ss, psem).wait()
            pltpu.make_async_copy(w_hbm.at[pl.ds(0, _SB)], sw, psem).wait()

        def _steps(sb, carry):
            ss, sd, sw, _ = sb

            def _step(i, carry):
                k, nf = carry
                sl0 = pl.ds(i * 32, 16)
                sl1 = pl.ds(i * 32 + 16, 16)
                d0 = sd[sl0]
                d1 = sd[sl1]
                inr0 = (d0 >= lo) & (d0 < hi)
                inr1 = (d1 >= lo) & (d1 < hi)
                ps0 = plsc.cumsum(jnp.where(inr0, _splat16(1), _splat16(0)))
                ps1 = plsc.cumsum(jnp.where(inr1, _splat16(1), _splat16(0)))
                t0 = ps0[15]
                pos0 = jnp.where(inr0, _splat16(k) + ps0 - 1, _splat16(_TRASH))
                pos1 = jnp.where(inr1, _splat16(k + t0) + ps1 - 1,
                                 _splat16(_TRASH))
                plsc.store_scatter(stage_dloc, [pos0], d0 - lo)
                plsc.store_scatter(stage_dloc, [pos1], d1 - lo)
                plsc.store_scatter(stage_src, [pos0], ss[sl0])
                plsc.store_scatter(stage_src, [pos1], ss[sl1])
                plsc.store_scatter(stage_w, [pos0], sw[sl0])
                plsc.store_scatter(stage_w, [pos1], sw[sl1])
                k = k + t0 + ps1[15]

                def _full(args):
                    k, nf = args
                    _flush_event(wh_hbm, nf)
                    return (k - 128, nf + 1)

                return lax.cond(k >= 128, _full, lambda a: a, (k, nf))

            return lax.fori_loop(0, _SB // 32, _step, carry)

        def _pair(bp, carry):
            base = e0 + bp * 2 * _SB
            _wait_pf(sbufs[0])
            _prefetch(sbufs[1], base + _SB)
            carry = _steps(sbufs[0], carry)
            _wait_pf(sbufs[1])
            nxt = jnp.minimum(base + 2 * _SB, e0 + pt - _SB)
            _prefetch(sbufs[0], nxt)
            carry = _steps(sbufs[1], carry)
            return carry

        _prefetch(sbufs[0], e0)
        k, nf = lax.fori_loop(0, pt // (2 * _SB), _pair, (0, 0))
        _wait_pf(sbufs[0])

        @pl.when(nf > 0)
        def _():
            lax.cond((nf - 1) % 2 == 0,
                     lambda: (_drain(wh_hbm, 0), 0)[1],
                     lambda: (_drain(wh_hbm, 1), 0)[1])

        @pl.when(nf >= 2)
        def _():
            lax.cond(nf % 2 == 0,
                     lambda: (_wait_scatter(0), 0)[1],
                     lambda: (_wait_scatter(1), 0)[1])

        @pl.when(nf >= 1)
        def _():
            lax.cond((nf - 1) % 2 == 0,
                     lambda: (_wait_scatter(0), 0)[1],
                     lambda: (_wait_scatter(1), 0)[1])

        for j in range(8):
            s2 = pl.ds(j * 16, 16)
            lanes = _splat16(j * 16) + _iota16()
            live = lanes < _splat16(k)
            dma_src0[s2] = jnp.where(live, stage_src[s2], _splat16(0))
            dma_dloc0[s2] = jnp.where(live, stage_dloc[s2], _splat16(_C))
            dma_w0[s2] = stage_w[s2]
        pltpu.sync_copy(wh_hbm.at[dma_src0], rows0)
        _scale(dma_w0, rows0)
        pltpu.sync_copy(rows0, feat_sh.at[dma_dloc0], add=True)
        pltpu.sync_copy(ones_v, cnt_sh.at[dma_dloc0], add=True)
        plsc.subcore_barrier()

        def _finblk(b, _):
            row0 = tile_row0 + b * 128
            gbase = sec_base + lo + row0
            local0 = lo + row0
            full = local0 + 128 <= n_valid
            pltpu.sync_copy(feat_sh.at[pl.ds(row0, 128)], rows0)
            pltpu.sync_copy(cnt_sh.at[pl.ds(row0, 128)], cbuf)
            if accumulate:
                def _rd_full():
                    pltpu.sync_copy(out_hbm.at[pl.ds(gbase, 128)], rows1)
                    return 0

                def _rd_part():
                    for q in range(16):
                        @pl.when(local0 + q * 8 < n_valid)
                        def _():
                            pltpu.sync_copy(
                                out_hbm.at[pl.ds(gbase + q * 8, 8)],
                                rows1.at[pl.ds(q * 8, 8)])
                    return 0
                lax.cond(full, _rd_full, _rd_part)

            def _fin(r2, _):
                r = r2 * 2
                c0 = plsc.load_gather(cbuf, [_splat16(r)])
                c1 = plsc.load_gather(cbuf, [_splat16(r + 1)])
                rc0 = jnp.where(c0 > 0, 1.0 / jnp.maximum(c0, 1.0), 0.0)
                rc1 = jnp.where(c1 > 0, 1.0 / jnp.maximum(c1, 1.0), 0.0)
                for j in range(8):
                    sl = pl.ds(j * 16, 16)
                    if accumulate:
                        rows0[r, sl] = rows0[r, sl] * rc0 + rows1[r, sl]
                    else:
                        rows0[r, sl] = rows0[r, sl] * rc0
                for j in range(8):
                    sl = pl.ds(j * 16, 16)
                    if accumulate:
                        rows0[r + 1, sl] = rows0[r + 1, sl] * rc1 + rows1[r + 1, sl]
                    else:
                        rows0[r + 1, sl] = rows0[r + 1, sl] * rc1
                return 0
            lax.fori_loop(0, 64, _fin, 0)

            def _wr_full():
                pltpu.sync_copy(rows0, out_hbm.at[pl.ds(gbase, 128)])
                return 0

            def _wr_part():
                for q in range(16):
                    @pl.when(local0 + q * 8 < n_valid)
                    def _():
                        pltpu.sync_copy(rows0.at[pl.ds(q * 8, 8)],
                                        out_hbm.at[pl.ds(gbase + q * 8, 8)])
                return 0
            lax.cond(full, _wr_full, _wr_part)
            return 0
        lax.fori_loop(0, rows_per_tile // 128, _finblk, 0)
        plsc.subcore_barrier()

    def _etype(wh_hbm, src_hbm, dst_hbm, w_hbm, nch, accumulate, mode,
               sec_base, n_valid):
        e_pad = src_hbm.shape[0]
        if mode == "split":
            trips = -(-nch // 2)

            def _chunk(ci, _):
                chunk = 2 * ci + cid

                @pl.when(chunk < nch)
                def _():
                    _pass(wh_hbm, src_hbm, dst_hbm, w_hbm, e_pad,
                          chunk * _C, accumulate, sec_base, n_valid)
                return 0
            lax.fori_loop(0, trips, _chunk, 0)
        else:
            core = 0 if mode == "core0" else 1

            @pl.when(cid == core)
            def _():
                def _chunk(ci, _):
                    _pass(wh_hbm, src_hbm, dst_hbm, w_hbm, e_pad,
                          ci * _C, accumulate, sec_base, n_valid)
                    return 0
                lax.fori_loop(0, nch, _chunk, 0)

    _etype(wh_ww, ww_src, ww_dst, ww_w, _NCH_W, False, "split", 0, NW)
    _etype(wh_wt, wt_src, wt_dst, wt_w, _NCH_T, False, "core0", NW, NT)
    _etype(wh_tt, tt_src, tt_dst, tt_w, _NCH_T, True, "core0", NW, NT)
    _etype(wh_wd, wd_src, wd_dst, wd_w, _NCH_D, False, "core1", NW + NT, ND)
    _etype(wh_td, td_src, td_dst, td_w, _NCH_D, True, "core1", NW + NT, ND)


def _sc_aggregate(wh_ww, wh_wt, wh_wd, wh_td, wh_tt, edges):
    (ww_src, ww_dst, ww_w, wt_src, wt_dst, wt_w, wd_src, wd_dst, wd_w,
     td_src, td_dst, td_w, tt_src, tt_dst, tt_w) = edges
    f32 = jnp.float32
    i32 = jnp.int32
    mesh = plsc.VectorSubcoreMesh(core_axis_name="c", subcore_axis_name="s")
    run = pl.kernel(
        _sc_agg_body,
        mesh=mesh,
        compiler_params=pltpu.CompilerParams(needs_layout_passes=False),
        out_type=[
            jax.ShapeDtypeStruct((NW + NT + ND, D), f32),
        ],
        scratch_types=[
            pltpu.VMEM_SHARED((_C + 8, D), f32),
            pltpu.VMEM_SHARED((_C + 8,), f32),
            pltpu.VMEM((_SB,), i32),
            pltpu.VMEM((_SB,), i32),
            pltpu.VMEM((_SB,), f32),
            pltpu.VMEM((_SB,), i32),
            pltpu.VMEM((_SB,), i32),
            pltpu.VMEM((_SB,), f32),
            pltpu.VMEM((_STG,), i32),
            pltpu.VMEM((_STG,), i32),
            pltpu.VMEM((_STG,), f32),
            pltpu.VMEM((128,), i32),
            pltpu.VMEM((128,), i32),
            pltpu.VMEM((128,), f32),
            pltpu.VMEM((128,), i32),
            pltpu.VMEM((128,), i32),
            pltpu.VMEM((128,), f32),
            pltpu.VMEM((128, D), f32),
            pltpu.VMEM((128, D), f32),
            pltpu.VMEM((128,), f32),
            pltpu.VMEM((128,), f32),
            pltpu.VMEM((128,), f32),
            pltpu.SemaphoreType.DMA,
            pltpu.SemaphoreType.DMA,
            pltpu.SemaphoreType.DMA,
            pltpu.SemaphoreType.DMA,
            pltpu.SemaphoreType.DMA,
            pltpu.SemaphoreType.DMA,
        ],
    )
    return run(wh_ww, wh_wt, wh_wd, wh_td, wh_tt,
               ww_src, ww_dst, ww_w, wt_src, wt_dst, wt_w,
               wd_src, wd_dst, wd_w, td_src, td_dst, td_w,
               tt_src, tt_dst, tt_w)


def kernel(feat_word, feat_topic, effect, ww_w, wt_w, wd_w, td_w, tt_w,
           W_ww, b_ww, W_wt, b_wt, W_wd, b_wd, W_td, b_td, W_tt, b_tt,
           W_cau, W_noi,
           ww_src, ww_dst, wt_src, wt_dst, wd_src, wd_dst,
           td_src, td_dst, tt_src, tt_dst):
    wh_ww, wh_wt, wh_wd = _word_proj(feat_word, W_ww, W_wt, W_wd,
                                     b_ww, b_wt, b_wd)
    wh_td, wh_tt = _topic_proj(feat_topic, effect, W_td, W_tt, W_cau, W_noi,
                               b_td, b_tt)
    flat = []
    for t in (_pad_edges(ww_src, ww_dst, ww_w),
              _pad_edges(wt_src, wt_dst, wt_w),
              _pad_edges(wd_src, wd_dst, wd_w),
              _pad_edges(td_src, td_dst, td_w),
              _pad_edges(tt_src, tt_dst, tt_w)):
        flat.extend(t)
    (out,) = _sc_aggregate(wh_ww, wh_wt, wh_wd, wh_td, wh_tt, tuple(flat))
    return out


# speedup vs baseline: 2.9427x; 1.0038x over previous
import functools

import jax
import jax.numpy as jnp
from jax import lax
from jax.experimental import pallas as pl
from jax.experimental.pallas import tpu as pltpu
from jax.experimental.pallas import tpu_sc as plsc

NW, NT, ND, D = 100000, 5000, 20000, 128
_BR = 2000
_C = 10240
_SB = 2048
_STG = 192
_TRASH = 184
_NSUB = 16

_NCH_W = -(-NW // _C)
_NCH_T = -(-NT // _C)
_NCH_D = -(-ND // _C)
_NWP = _NCH_W * _C
_NTP = _NCH_T * _C
_NDP = _NCH_D * _C


def _word_proj_body(x_ref, www_ref, wwt_ref, wwd_ref, bww_ref, bwt_ref,
                    bwd_ref, o1_ref, o2_ref, o3_ref):
    x = x_ref[...]
    o1_ref[...] = jnp.dot(x, www_ref[...], preferred_element_type=jnp.float32) + bww_ref[...]
    o2_ref[...] = jnp.dot(x, wwt_ref[...], preferred_element_type=jnp.float32) + bwt_ref[...]
    o3_ref[...] = jnp.dot(x, wwd_ref[...], preferred_element_type=jnp.float32) + bwd_ref[...]


def _topic_proj_body(ft_ref, eff_ref, wtd_ref, wtt_ref, wcau_ref, wnoi_ref,
                     btd_ref, btt_ref, otd_ref, ott_ref):
    ft = ft_ref[...]
    eff = eff_ref[...]
    pos = (eff > 0).astype(jnp.float32)
    neg = (eff < 0).astype(jnp.float32)
    cau = jnp.dot(ft * pos, wcau_ref[...], preferred_element_type=jnp.float32)
    noi = jnp.dot(ft * neg, wnoi_ref[...], preferred_element_type=jnp.float32)
    cmn = cau - noi
    otd_ref[...] = jnp.dot(ft, wtd_ref[...], preferred_element_type=jnp.float32) + btd_ref[...] + cmn
    ott_ref[...] = jnp.dot(ft, wtt_ref[...], preferred_element_type=jnp.float32) + btt_ref[...] + cmn


def _word_proj(feat_word, W_ww, W_wt, W_wd, b_ww, b_wt, b_wd):
    n = feat_word.shape[0]
    blk = pl.BlockSpec((_BR, D), lambda i: (i, 0))
    wblk = pl.BlockSpec((D, D), lambda i: (0, 0))
    bblk = pl.BlockSpec((1, D), lambda i: (0, 0))
    return pl.pallas_call(
        _word_proj_body,
        grid=(n // _BR,),
        in_specs=[blk, wblk, wblk, wblk, bblk, bblk, bblk],
        out_specs=[blk, blk, blk],
        out_shape=[jax.ShapeDtypeStruct((n, D), jnp.float32)] * 3,
    )(feat_word, W_ww, W_wt, W_wd, b_ww.reshape(1, D), b_wt.reshape(1, D),
      b_wd.reshape(1, D))


def _topic_proj(feat_topic, effect, W_td, W_tt, W_cau, W_noi, b_td, b_tt):
    n = feat_topic.shape[0]
    return pl.pallas_call(
        _topic_proj_body,
        out_shape=[jax.ShapeDtypeStruct((n, D), jnp.float32)] * 2,
    )(feat_topic, effect, W_td, W_tt, W_cau, W_noi, b_td.reshape(1, D),
      b_tt.reshape(1, D))


def _pad_edges(src, dst, w):
    e = src.shape[0]
    unit = _NSUB * _SB * 2
    e_pad = unit * (-(-e // unit))
    pad = e_pad - e
    return (jnp.pad(src, (0, pad)),
            jnp.pad(dst, (0, pad), constant_values=-1),
            jnp.pad(w, (0, pad)))


def _iota16():
    return lax.iota(jnp.int32, 16)


def _splat16(x):
    return jnp.full((16,), x, jnp.int32)


def _sc_agg_body(
    wh_ww, wh_wt, wh_wd, wh_td, wh_tt,
    ww_src, ww_dst, ww_w, wt_src, wt_dst, wt_w, wd_src, wd_dst, wd_w,
    td_src, td_dst, td_w, tt_src, tt_dst, tt_w,
    out_hbm,
    feat_sh, cnt_sh,
    scan_src0, scan_dst0, scan_w0, scan_src1, scan_dst1, scan_w1,
    stage_src, stage_dloc, stage_w,
    dma_src0, dma_dloc0, dma_w0, dma_src1, dma_dloc1, dma_w1,
    rows0, rows1, cbuf, ones_v, zc_v,
    gsem0, gsem1, ssem0, ssem1, psem0, psem1,
):
    cid = lax.axis_index("c")
    sid = lax.axis_index("s")

    for j in range(8):
        ones_v[pl.ds(j * 16, 16)] = jnp.ones((16,), jnp.float32)
        zc_v[pl.ds(j * 16, 16)] = jnp.zeros((16,), jnp.float32)

    rows_per_tile = _C // _NSUB
    tile_row0 = sid * rows_per_tile
    bufs = ((dma_src0, dma_dloc0, dma_w0, rows0, gsem0, ssem0),
            (dma_src1, dma_dloc1, dma_w1, rows1, gsem1, ssem1))
    sbufs = ((scan_src0, scan_dst0, scan_w0, psem0),
             (scan_src1, scan_dst1, scan_w1, psem1))

    def _fill(dma_s, dma_d, dma_wt):
        for j in range(8):
            s2 = pl.ds(j * 16, 16)
            dma_s[s2] = stage_src[s2]
            dma_d[s2] = stage_dloc[s2]
            dma_wt[s2] = stage_w[s2]

    def _scale(dma_wt, rws):
        def _srow(r2, _):
            r = r2 * 2
            wv0 = plsc.load_gather(dma_wt, [_splat16(r)])
            wv1 = plsc.load_gather(dma_wt, [_splat16(r + 1)])
            for j in range(8):
                sl = pl.ds(j * 16, 16)
                rws[r, sl] = rws[r, sl] * wv0
            for j in range(8):
                sl = pl.ds(j * 16, 16)
                rws[r + 1, sl] = rws[r + 1, sl] * wv1
            return 0
        lax.fori_loop(0, 64, _srow, 0)

    def _drain(wh_hbm, par):
        dma_s, dma_d, dma_wt, rws, gsem, ssem = bufs[par]
        pltpu.make_async_copy(wh_hbm.at[dma_s], rws, gsem).wait()
        _scale(dma_wt, rws)
        pltpu.async_copy(rws, feat_sh.at[dma_d], ssem, add=True)
        pltpu.async_copy(ones_v, cnt_sh.at[dma_d], ssem, add=True)

    def _wait_scatter(par):
        dma_s, dma_d, dma_wt, rws, gsem, ssem = bufs[par]
        pltpu.make_async_copy(rws, feat_sh.at[dma_d], ssem).wait()
        pltpu.make_async_copy(ones_v, cnt_sh.at[dma_d], ssem).wait()

    def _flush_event(wh_hbm, nf):
        def _go(par):
            dma_s, dma_d, dma_wt, rws, gsem, ssem = bufs[par]

            @pl.when(nf >= 2)
            def _():
                _wait_scatter(par)
            _fill(dma_s, dma_d, dma_wt)
            pltpu.async_copy(wh_hbm.at[dma_s], rws, gsem)

            @pl.when(nf > 0)
            def _():
                _drain(wh_hbm, 1 - par)
            return 0
        lax.cond(nf % 2 == 0, lambda: _go(0), lambda: _go(1))
        for t in range(2):
            stage_src[pl.ds(t * 16, 16)] = stage_src[pl.ds(128 + t * 16, 16)]
            stage_dloc[pl.ds(t * 16, 16)] = stage_dloc[pl.ds(128 + t * 16, 16)]
            stage_w[pl.ds(t * 16, 16)] = stage_w[pl.ds(128 + t * 16, 16)]

    def _pass(wh_hbm, src_hbm, dst_hbm, w_hbm, e_pad, lo, accumulate,
              sec_base, n_valid):
        def _zrow(r, _):
            for j in range(8):
                rows0[r, pl.ds(j * 16, 16)] = jnp.zeros((16,), jnp.float32)
            return 0
        lax.fori_loop(0, 128, _zrow, 0)
        for b in range(rows_per_tile // 128):
            pltpu.sync_copy(rows0, feat_sh.at[pl.ds(tile_row0 + b * 128, 128)])
            pltpu.sync_copy(zc_v, cnt_sh.at[pl.ds(tile_row0 + b * 128, 128)])
        plsc.subcore_barrier()

        pt = e_pad // _NSUB
        e0 = sid * pt
        hi = lo + _C

        def _prefetch(sb, pos0):
            ss, sd, sw, psem = sb
            pltpu.async_copy(dst_hbm.at[pl.ds(pos0, _SB)], sd, psem)
            pltpu.async_copy(src_hbm.at[pl.ds(pos0, _SB)], ss, psem)
            pltpu.async_copy(w_hbm.at[pl.ds(pos0, _SB)], sw, psem)

        def _wait_pf(sb):
            ss, sd, sw, psem = sb
            pltpu.make_async_copy(dst_hbm.at[pl.ds(0, _SB)], sd, psem).wait()
            pltpu.make_async_copy(src_hbm.at[pl.ds(0, _SB)], ss, psem).wait()
            pltpu.make_async_copy(w_hbm.at[pl.ds(0, _SB)], sw, psem).wait()

        def _steps(sb, carry):
            ss, sd, sw, _ = sb

            def _step(i, carry):
                k, nf = carry
                sl0 = pl.ds(i * 32, 16)
                sl1 = pl.ds(i * 32 + 16, 16)
                d0 = sd[sl0]
                d1 = sd[sl1]
                inr0 = (d0 >= lo) & (d0 < hi)
                inr1 = (d1 >= lo) & (d1 < hi)
                ps0 = plsc.cumsum(jnp.where(inr0, _splat16(1), _splat16(0)))
                ps1 = plsc.cumsum(jnp.where(inr1, _splat16(1), _splat16(0)))
                t0 = ps0[15]
                pos0 = jnp.where(inr0, _splat16(k) + ps0 - 1, _splat16(_TRASH))
                pos1 = jnp.where(inr1, _splat16(k + t0) + ps1 - 1,
                                 _splat16(_TRASH))
                plsc.store_scatter(stage_dloc, [pos0], d0 - lo)
                plsc.store_scatter(stage_dloc, [pos1], d1 - lo)
                plsc.store_scatter(stage_src, [pos0], ss[sl0])
                plsc.store_scatter(stage_src, [pos1], ss[sl1])
                plsc.store_scatter(stage_w, [pos0], sw[sl0])
                plsc.store_scatter(stage_w, [pos1], sw[sl1])
                k = k + t0 + ps1[15]

                def _full(args):
                    k, nf = args
                    _flush_event(wh_hbm, nf)
                    return (k - 128, nf + 1)

                return lax.cond(k >= 128, _full, lambda a: a, (k, nf))

            return lax.fori_loop(0, _SB // 32, _step, carry)

        def _pair(bp, carry):
            base = e0 + bp * 2 * _SB
            _wait_pf(sbufs[0])
            _prefetch(sbufs[1], base + _SB)
            carry = _steps(sbufs[0], carry)
            _wait_pf(sbufs[1])
            nxt = jnp.minimum(base + 2 * _SB, e0 + pt - _SB)
            _prefetch(sbufs[0], nxt)
            carry = _steps(sbufs[1], carry)
            return carry

        _prefetch(sbufs[0], e0)
        k, nf = lax.fori_loop(0, pt // (2 * _SB), _pair, (0, 0))
        _wait_pf(sbufs[0])

        @pl.when(nf > 0)
        def _():
            lax.cond((nf - 1) % 2 == 0,
                     lambda: (_drain(wh_hbm, 0), 0)[1],
                     lambda: (_drain(wh_hbm, 1), 0)[1])

        @pl.when(nf >= 2)
        def _():
            lax.cond(nf % 2 == 0,
                     lambda: (_wait_scatter(0), 0)[1],
                     lambda: (_wait_scatter(1), 0)[1])

        @pl.when(nf >= 1)
        def _():
            lax.cond((nf - 1) % 2 == 0,
                     lambda: (_wait_scatter(0), 0)[1],
                     lambda: (_wait_scatter(1), 0)[1])

        for j in range(8):
            s2 = pl.ds(j * 16, 16)
            lanes = _splat16(j * 16) + _iota16()
            live = lanes < _splat16(k)
            dma_src0[s2] = jnp.where(live, stage_src[s2], _splat16(0))
            dma_dloc0[s2] = jnp.where(live, stage_dloc[s2], _splat16(_C))
            dma_w0[s2] = stage_w[s2]
        pltpu.sync_copy(wh_hbm.at[dma_src0], rows0)
        _scale(dma_w0, rows0)
        pltpu.sync_copy(rows0, feat_sh.at[dma_dloc0], add=True)
        pltpu.sync_copy(ones_v, cnt_sh.at[dma_dloc0], add=True)
        plsc.subcore_barrier()

        def _finblk(b, _):
            row0 = tile_row0 + b * 128
            gbase = sec_base + lo + row0
            local0 = lo + row0
            full = local0 + 128 <= n_valid
            pltpu.sync_copy(feat_sh.at[pl.ds(row0, 128)], rows0)
            pltpu.sync_copy(cnt_sh.at[pl.ds(row0, 128)], cbuf)
            if accumulate:
                def _rd_full():
                    pltpu.sync_copy(out_hbm.at[pl.ds(gbase, 128)], rows1)
                    return 0

                def _rd_part():
                    for q in range(16):
                        @pl.when(local0 + q * 8 < n_valid)
                        def _():
                            pltpu.sync_copy(
                                out_hbm.at[pl.ds(gbase + q * 8, 8)],
                                rows1.at[pl.ds(q * 8, 8)])
                    return 0
                lax.cond(full, _rd_full, _rd_part)

            def _fin(r2, _):
                r = r2 * 2
                c0 = plsc.load_gather(cbuf, [_splat16(r)])
                c1 = plsc.load_gather(cbuf, [_splat16(r + 1)])
                rc0 = jnp.where(c0 > 0, 1.0 / jnp.maximum(c0, 1.0), 0.0)
                rc1 = jnp.where(c1 > 0, 1.0 / jnp.maximum(c1, 1.0), 0.0)
                for j in range(8):
                    sl = pl.ds(j * 16, 16)
                    if accumulate:
                        rows0[r, sl] = rows0[r, sl] * rc0 + rows1[r, sl]
                    else:
                        rows0[r, sl] = rows0[r, sl] * rc0
                for j in range(8):
                    sl = pl.ds(j * 16, 16)
                    if accumulate:
                        rows0[r + 1, sl] = rows0[r + 1, sl] * rc1 + rows1[r + 1, sl]
                    else:
                        rows0[r + 1, sl] = rows0[r + 1, sl] * rc1
                return 0
            lax.fori_loop(0, 64, _fin, 0)

            def _wr_full():
                pltpu.sync_copy(rows0, out_hbm.at[pl.ds(gbase, 128)])
                return 0

            def _wr_part():
                for q in range(16):
                    @pl.when(local0 + q * 8 < n_valid)
                    def _():
                        pltpu.sync_copy(rows0.at[pl.ds(q * 8, 8)],
                                        out_hbm.at[pl.ds(gbase + q * 8, 8)])
                return 0
            lax.cond(full, _wr_full, _wr_part)
            return 0
        lax.fori_loop(0, rows_per_tile // 128, _finblk, 0)
        plsc.subcore_barrier()

    def _etype(wh_hbm, src_hbm, dst_hbm, w_hbm, nch, accumulate, mode,
               sec_base, n_valid):
        e_pad = src_hbm.shape[0]
        if mode == "split":
            trips = -(-nch // 2)

            def _chunk(ci, _):
                chunk = 2 * ci + cid

                @pl.when(chunk < nch)
                def _():
                    _pass(wh_hbm, src_hbm, dst_hbm, w_hbm, e_pad,
                          chunk * _C, accumulate, sec_base, n_valid)
                return 0
            lax.fori_loop(0, trips, _chunk, 0)
        else:
            core = 0 if mode == "core0" else 1

            @pl.when(cid == core)
            def _():
                def _chunk(ci, _):
                    _pass(wh_hbm, src_hbm, dst_hbm, w_hbm, e_pad,
                          ci * _C, accumulate, sec_base, n_valid)
                    return 0
                lax.fori_loop(0, nch, _chunk, 0)

    _etype(wh_ww, ww_src, ww_dst, ww_w, _NCH_W, False, "split", 0, NW)
    _etype(wh_wt, wt_src, wt_dst, wt_w, _NCH_T, False, "core0", NW, NT)
    _etype(wh_tt, tt_src, tt_dst, tt_w, _NCH_T, True, "core0", NW, NT)
    _etype(wh_wd, wd_src, wd_dst, wd_w, _NCH_D, False, "core1", NW + NT, ND)
    _etype(wh_td, td_src, td_dst, td_w, _NCH_D, True, "core1", NW + NT, ND)


def _sc_aggregate(wh_ww, wh_wt, wh_wd, wh_td, wh_tt, edges):
    (ww_src, ww_dst, ww_w, wt_src, wt_dst, wt_w, wd_src, wd_dst, wd_w,
     td_src, td_dst, td_w, tt_src, tt_dst, tt_w) = edges
    f32 = jnp.float32
    i32 = jnp.int32
    mesh = plsc.VectorSubcoreMesh(core_axis_name="c", subcore_axis_name="s")
    run = pl.kernel(
        _sc_agg_body,
        mesh=mesh,
        compiler_params=pltpu.CompilerParams(needs_layout_passes=False),
        out_type=[
            jax.ShapeDtypeStruct((NW + NT + ND, D), f32),
        ],
        scratch_types=[
            pltpu.VMEM_SHARED((_C + 8, D), f32),
            pltpu.VMEM_SHARED((_C + 8,), f32),
            pltpu.VMEM((_SB,), i32),
            pltpu.VMEM((_SB,), i32),
            pltpu.VMEM((_SB,), f32),
            pltpu.VMEM((_SB,), i32),
            pltpu.VMEM((_SB,), i32),
            pltpu.VMEM((_SB,), f32),
            pltpu.VMEM((_STG,), i32),
            pltpu.VMEM((_STG,), i32),
            pltpu.VMEM((_STG,), f32),
            pltpu.VMEM((128,), i32),
            pltpu.VMEM((128,), i32),
            pltpu.VMEM((128,), f32),
            pltpu.VMEM((128,), i32),
            pltpu.VMEM((128,), i32),
            pltpu.VMEM((128,), f32),
            pltpu.VMEM((128, D), f32),
            pltpu.VMEM((128, D), f32),
            pltpu.VMEM((128,), f32),
            pltpu.VMEM((128,), f32),
            pltpu.VMEM((128,), f32),
            pltpu.SemaphoreType.DMA,
            pltpu.SemaphoreType.DMA,
            pltpu.SemaphoreType.DMA,
            pltpu.SemaphoreType.DMA,
            pltpu.SemaphoreType.DMA,
            pltpu.SemaphoreType.DMA,
        ],
    )
    return run(wh_ww, wh_wt, wh_wd, wh_td, wh_tt,
               ww_src, ww_dst, ww_w, wt_src, wt_dst, wt_w,
               wd_src, wd_dst, wd_w, td_src, td_dst, td_w,
               tt_src, tt_dst, tt_w)


def kernel(feat_word, feat_topic, effect, ww_w, wt_w, wd_w, td_w, tt_w,
           W_ww, b_ww, W_wt, b_wt, W_wd, b_wd, W_td, b_td, W_tt, b_tt,
           W_cau, W_noi,
           ww_src, ww_dst, wt_src, wt_dst, wd_src, wd_dst,
           td_src, td_dst, tt_src, tt_dst):
    wh_ww, wh_wt, wh_wd = _word_proj(feat_word, W_ww, W_wt, W_wd,
                                     b_ww, b_wt, b_wd)
    wh_td, wh_tt = _topic_proj(feat_topic, effect, W_td, W_tt, W_cau, W_noi,
                               b_td, b_tt)
    flat = []
    for t in (_pad_edges(ww_src, ww_dst, ww_w),
              _pad_edges(wt_src, wt_dst, wt_w),
              _pad_edges(wd_src, wd_dst, wd_w),
              _pad_edges(td_src, td_dst, td_w),
              _pad_edges(tt_src, tt_dst, tt_w)):
        flat.extend(t)
    (out,) = _sc_aggregate(wh_ww, wh_wt, wh_wd, wh_td, wh_tt, tuple(flat))
    return out


# speedup vs baseline: 2.9444x; 1.0006x over previous
import jax
import jax.numpy as jnp
from jax import lax
from jax.experimental import pallas as pl
from jax.experimental.pallas import tpu as pltpu
from jax.experimental.pallas import tpu_sc as plsc

NW, NT, ND, D = 100000, 5000, 20000, 128
_BR = 2000
_C = 10240
_SB = 2048
_STG = 192
_TRASH = 184
_NSUB = 16

_NCH_W = -(-NW // _C)
_NCH_T = -(-NT // _C)
_NCH_D = -(-ND // _C)


def _word_proj_body(x_ref, www_ref, wwt_ref, wwd_ref, bww_ref, bwt_ref,
                    bwd_ref, o1_ref, o2_ref, o3_ref):
    x = x_ref[...]
    o1_ref[...] = jnp.dot(x, www_ref[...], preferred_element_type=jnp.float32) + bww_ref[...]
    o2_ref[...] = jnp.dot(x, wwt_ref[...], preferred_element_type=jnp.float32) + bwt_ref[...]
    o3_ref[...] = jnp.dot(x, wwd_ref[...], preferred_element_type=jnp.float32) + bwd_ref[...]


def _topic_proj_body(ft_ref, eff_ref, wtd_ref, wtt_ref, wcau_ref, wnoi_ref,
                     btd_ref, btt_ref, otd_ref, ott_ref):
    ft = ft_ref[...]
    eff = eff_ref[...]
    pos = (eff > 0).astype(jnp.float32)
    neg = (eff < 0).astype(jnp.float32)
    cau = jnp.dot(ft * pos, wcau_ref[...], preferred_element_type=jnp.float32)
    noi = jnp.dot(ft * neg, wnoi_ref[...], preferred_element_type=jnp.float32)
    cmn = cau - noi
    otd_ref[...] = jnp.dot(ft, wtd_ref[...], preferred_element_type=jnp.float32) + btd_ref[...] + cmn
    ott_ref[...] = jnp.dot(ft, wtt_ref[...], preferred_element_type=jnp.float32) + btt_ref[...] + cmn


def _word_proj(feat_word, W_ww, W_wt, W_wd, b_ww, b_wt, b_wd):
    n = feat_word.shape[0]
    blk = pl.BlockSpec((_BR, D), lambda i: (i, 0))
    wblk = pl.BlockSpec((D, D), lambda i: (0, 0))
    bblk = pl.BlockSpec((1, D), lambda i: (0, 0))
    return pl.pallas_call(
        _word_proj_body,
        grid=(n // _BR,),
        in_specs=[blk, wblk, wblk, wblk, bblk, bblk, bblk],
        out_specs=[blk, blk, blk],
        out_shape=[jax.ShapeDtypeStruct((n, D), jnp.float32)] * 3,
    )(feat_word, W_ww, W_wt, W_wd, b_ww.reshape(1, D), b_wt.reshape(1, D),
      b_wd.reshape(1, D))


def _topic_proj(feat_topic, effect, W_td, W_tt, W_cau, W_noi, b_td, b_tt):
    n = feat_topic.shape[0]
    return pl.pallas_call(
        _topic_proj_body,
        out_shape=[jax.ShapeDtypeStruct((n, D), jnp.float32)] * 2,
    )(feat_topic, effect, W_td, W_tt, W_cau, W_noi, b_td.reshape(1, D),
      b_tt.reshape(1, D))


def _pad_edges(src, dst, w):
    e = src.shape[0]
    unit = _NSUB * _SB * 2
    e_pad = unit * (-(-e // unit))
    pad = e_pad - e
    return (jnp.pad(src, (0, pad)),
            jnp.pad(dst, (0, pad), constant_values=-1),
            jnp.pad(w, (0, pad)))


def _iota16():
    return lax.iota(jnp.int32, 16)


def _splat16(x):
    return jnp.full((16,), x, jnp.int32)


def _sc_agg_body(
    wh_ww, wh_wt, wh_wd, wh_td, wh_tt,
    ww_src, ww_dst, ww_w, wt_src, wt_dst, wt_w, wd_src, wd_dst, wd_w,
    td_src, td_dst, td_w, tt_src, tt_dst, tt_w,
    out_hbm,
    feat_sh, cnt_sh,
    scan_src0, scan_dst0, scan_w0, scan_src1, scan_dst1, scan_w1,
    stage_src, stage_dloc, stage_w,
    dma_src0, dma_dloc0, dma_w0, dma_src1, dma_dloc1, dma_w1,
    rows0, rows1, cbuf, ones_v, zc_v,
    gsem0, gsem1, ssem0, ssem1, psem0, psem1,
):
    cid = lax.axis_index("c")
    sid = lax.axis_index("s")

    for j in range(8):
        ones_v[pl.ds(j * 16, 16)] = jnp.ones((16,), jnp.float32)

    def _zc_init(t, _):
        zc_v[pl.ds(t * 16, 16)] = jnp.zeros((16,), jnp.float32)
        return 0
    lax.fori_loop(0, _C // _NSUB // 16, _zc_init, 0)

    rows_per_tile = _C // _NSUB
    tile_row0 = sid * rows_per_tile
    bufs = ((dma_src0, dma_dloc0, dma_w0, rows0, gsem0, ssem0),
            (dma_src1, dma_dloc1, dma_w1, rows1, gsem1, ssem1))
    sbufs = ((scan_src0, scan_dst0, scan_w0, psem0),
             (scan_src1, scan_dst1, scan_w1, psem1))

    def _fill(dma_s, dma_d, dma_wt):
        for j in range(8):
            s2 = pl.ds(j * 16, 16)
            dma_s[s2] = stage_src[s2]
            dma_d[s2] = stage_dloc[s2]
            dma_wt[s2] = stage_w[s2]

    def _scale(dma_wt, rws):
        def _srow(r2, _):
            r = r2 * 2
            wv0 = plsc.load_gather(dma_wt, [_splat16(r)])
            wv1 = plsc.load_gather(dma_wt, [_splat16(r + 1)])
            for j in range(8):
                sl = pl.ds(j * 16, 16)
                rws[r, sl] = rws[r, sl] * wv0
            for j in range(8):
                sl = pl.ds(j * 16, 16)
                rws[r + 1, sl] = rws[r + 1, sl] * wv1
            return 0
        lax.fori_loop(0, 64, _srow, 0)

    def _drain(wh_hbm, par):
        dma_s, dma_d, dma_wt, rws, gsem, ssem = bufs[par]
        pltpu.make_async_copy(wh_hbm.at[dma_s], rws, gsem).wait()
        _scale(dma_wt, rws)
        pltpu.async_copy(rws, feat_sh.at[dma_d], ssem, add=True)
        pltpu.async_copy(ones_v, cnt_sh.at[dma_d], ssem, add=True)

    def _wait_scatter(par):
        dma_s, dma_d, dma_wt, rws, gsem, ssem = bufs[par]
        pltpu.make_async_copy(rws, feat_sh.at[dma_d], ssem).wait()
        pltpu.make_async_copy(ones_v, cnt_sh.at[dma_d], ssem).wait()

    def _flush_event(wh_hbm, nf):
        def _go(par):
            dma_s, dma_d, dma_wt, rws, gsem, ssem = bufs[par]

            @pl.when(nf >= 2)
            def _():
                _wait_scatter(par)
            _fill(dma_s, dma_d, dma_wt)
            pltpu.async_copy(wh_hbm.at[dma_s], rws, gsem)

            @pl.when(nf > 0)
            def _():
                _drain(wh_hbm, 1 - par)
            return 0
        lax.cond(nf % 2 == 0, lambda: _go(0), lambda: _go(1))
        for t in range(2):
            stage_src[pl.ds(t * 16, 16)] = stage_src[pl.ds(128 + t * 16, 16)]
            stage_dloc[pl.ds(t * 16, 16)] = stage_dloc[pl.ds(128 + t * 16, 16)]
            stage_w[pl.ds(t * 16, 16)] = stage_w[pl.ds(128 + t * 16, 16)]

    def _pass(wh_hbm, src_hbm, dst_hbm, w_hbm, e_pad, lo, accumulate,
              sec_base, n_valid):
        def _zrow(r, _):
            for j in range(8):
                rows0[r, pl.ds(j * 16, 16)] = jnp.zeros((16,), jnp.float32)
            return 0
        lax.fori_loop(0, 128, _zrow, 0)
        for b in range(rows_per_tile // 128):
            @pl.when(lo + tile_row0 + b * 128 < n_valid)
            def _():
                pltpu.sync_copy(rows0,
                                feat_sh.at[pl.ds(tile_row0 + b * 128, 128)])
        pltpu.sync_copy(zc_v, cnt_sh.at[pl.ds(tile_row0, rows_per_tile)])
        plsc.subcore_barrier()

        pt = e_pad // _NSUB
        e0 = sid * pt
        hi = lo + _C

        def _prefetch(sb, pos0):
            ss, sd, sw, psem = sb
            pltpu.async_copy(dst_hbm.at[pl.ds(pos0, _SB)], sd, psem)
            pltpu.async_copy(src_hbm.at[pl.ds(pos0, _SB)], ss, psem)
            pltpu.async_copy(w_hbm.at[pl.ds(pos0, _SB)], sw, psem)

        def _wait_pf(sb):
            ss, sd, sw, psem = sb
            pltpu.make_async_copy(dst_hbm.at[pl.ds(0, _SB)], sd, psem).wait()
            pltpu.make_async_copy(src_hbm.at[pl.ds(0, _SB)], ss, psem).wait()
            pltpu.make_async_copy(w_hbm.at[pl.ds(0, _SB)], sw, psem).wait()

        def _steps(sb, carry):
            ss, sd, sw, _ = sb

            def _step(i, carry):
                k, nf = carry
                sl0 = pl.ds(i * 32, 16)
                sl1 = pl.ds(i * 32 + 16, 16)
                d0 = sd[sl0]
                d1 = sd[sl1]
                inr0 = (d0 >= lo) & (d0 < hi)
                inr1 = (d1 >= lo) & (d1 < hi)
                ps0 = plsc.cumsum(jnp.where(inr0, _splat16(1), _splat16(0)))
                ps1 = plsc.cumsum(jnp.where(inr1, _splat16(1), _splat16(0)))
                t0 = ps0[15]
                pos0 = jnp.where(inr0, _splat16(k) + ps0 - 1, _splat16(_TRASH))
                pos1 = jnp.where(inr1, _splat16(k + t0) + ps1 - 1,
                                 _splat16(_TRASH))
                plsc.store_scatter(stage_dloc, [pos0], d0 - lo)
                plsc.store_scatter(stage_dloc, [pos1], d1 - lo)
                plsc.store_scatter(stage_src, [pos0], ss[sl0])
                plsc.store_scatter(stage_src, [pos1], ss[sl1])
                plsc.store_scatter(stage_w, [pos0], sw[sl0])
                plsc.store_scatter(stage_w, [pos1], sw[sl1])
                k = k + t0 + ps1[15]

                def _full(args):
                    k, nf = args
                    _flush_event(wh_hbm, nf)
                    return (k - 128, nf + 1)

                return lax.cond(k >= 128, _full, lambda a: a, (k, nf))

            return lax.fori_loop(0, _SB // 32, _step, carry)

        def _pair(bp, carry):
            base = e0 + bp * 2 * _SB
            _wait_pf(sbufs[0])
            _prefetch(sbufs[1], base + _SB)
            carry = _steps(sbufs[0], carry)
            _wait_pf(sbufs[1])
            nxt = jnp.minimum(base + 2 * _SB, e0 + pt - _SB)
            _prefetch(sbufs[0], nxt)
            carry = _steps(sbufs[1], carry)
            return carry

        _prefetch(sbufs[0], e0)
        k, nf = lax.fori_loop(0, pt // (2 * _SB), _pair, (0, 0))
        _wait_pf(sbufs[0])

        @pl.when(nf > 0)
        def _():
            lax.cond((nf - 1) % 2 == 0,
                     lambda: (_drain(wh_hbm, 0), 0)[1],
                     lambda: (_drain(wh_hbm, 1), 0)[1])

        @pl.when(nf >= 2)
        def _():
            lax.cond(nf % 2 == 0,
                     lambda: (_wait_scatter(0), 0)[1],
                     lambda: (_wait_scatter(1), 0)[1])

        @pl.when(nf >= 1)
        def _():
            lax.cond((nf - 1) % 2 == 0,
                     lambda: (_wait_scatter(0), 0)[1],
                     lambda: (_wait_scatter(1), 0)[1])

        for j in range(8):
            s2 = pl.ds(j * 16, 16)
            lanes = _splat16(j * 16) + _iota16()
            live = lanes < _splat16(k)
            dma_src0[s2] = jnp.where(live, stage_src[s2], _splat16(0))
            dma_dloc0[s2] = jnp.where(live, stage_dloc[s2], _splat16(_C))
            dma_w0[s2] = stage_w[s2]
        pltpu.sync_copy(wh_hbm.at[dma_src0], rows0)
        _scale(dma_w0, rows0)
        pltpu.sync_copy(rows0, feat_sh.at[dma_dloc0], add=True)
        pltpu.sync_copy(ones_v, cnt_sh.at[dma_dloc0], add=True)
        plsc.subcore_barrier()

        def _finblk(b, _):
            row0 = tile_row0 + b * 128
            gbase = sec_base + lo + row0
            local0 = lo + row0
            full = local0 + 128 <= n_valid

            @pl.when(local0 < n_valid)
            def _():
                _finblk_live(row0, gbase, local0, full)
            return 0

        def _finblk_live(row0, gbase, local0, full):
            pltpu.sync_copy(feat_sh.at[pl.ds(row0, 128)], rows0)
            pltpu.sync_copy(cnt_sh.at[pl.ds(row0, 128)], cbuf)
            if accumulate:
                def _rd_full():
                    pltpu.sync_copy(out_hbm.at[pl.ds(gbase, 128)], rows1)
                    return 0

                def _rd_part():
                    for q in range(16):
                        @pl.when(local0 + q * 8 < n_valid)
                        def _():
                            pltpu.sync_copy(
                                out_hbm.at[pl.ds(gbase + q * 8, 8)],
                                rows1.at[pl.ds(q * 8, 8)])
                    return 0
                lax.cond(full, _rd_full, _rd_part)

            def _fin(r2, _):
                r = r2 * 2
                c0 = plsc.load_gather(cbuf, [_splat16(r)])
                c1 = plsc.load_gather(cbuf, [_splat16(r + 1)])
                rc0 = jnp.where(c0 > 0, 1.0 / jnp.maximum(c0, 1.0), 0.0)
                rc1 = jnp.where(c1 > 0, 1.0 / jnp.maximum(c1, 1.0), 0.0)
                for j in range(8):
                    sl = pl.ds(j * 16, 16)
                    if accumulate:
                        rows0[r, sl] = rows0[r, sl] * rc0 + rows1[r, sl]
                    else:
                        rows0[r, sl] = rows0[r, sl] * rc0
                for j in range(8):
                    sl = pl.ds(j * 16, 16)
                    if accumulate:
                        rows0[r + 1, sl] = rows0[r + 1, sl] * rc1 + rows1[r + 1, sl]
                    else:
                        rows0[r + 1, sl] = rows0[r + 1, sl] * rc1
                return 0
            lax.fori_loop(0, 64, _fin, 0)

            def _wr_full():
                pltpu.sync_copy(rows0, out_hbm.at[pl.ds(gbase, 128)])
                return 0

            def _wr_part():
                for q in range(16):
                    @pl.when(local0 + q * 8 < n_valid)
                    def _():
                        pltpu.sync_copy(rows0.at[pl.ds(q * 8, 8)],
                                        out_hbm.at[pl.ds(gbase + q * 8, 8)])
                return 0
            lax.cond(full, _wr_full, _wr_part)
        lax.fori_loop(0, rows_per_tile // 128, _finblk, 0)
        plsc.subcore_barrier()

    def _etype(wh_hbm, src_hbm, dst_hbm, w_hbm, nch, accumulate, mode,
               sec_base, n_valid):
        e_pad = src_hbm.shape[0]
        if mode == "split":
            trips = -(-nch // 2)

            def _chunk(ci, _):
                chunk = 2 * ci + cid

                @pl.when(chunk < nch)
                def _():
                    _pass(wh_hbm, src_hbm, dst_hbm, w_hbm, e_pad,
                          chunk * _C, accumulate, sec_base, n_valid)
                return 0
            lax.fori_loop(0, trips, _chunk, 0)
        else:
            core = 0 if mode == "core0" else 1

            @pl.when(cid == core)
            def _():
                def _chunk(ci, _):
                    _pass(wh_hbm, src_hbm, dst_hbm, w_hbm, e_pad,
                          ci * _C, accumulate, sec_base, n_valid)
                    return 0
                lax.fori_loop(0, nch, _chunk, 0)

    _etype(wh_ww, ww_src, ww_dst, ww_w, _NCH_W, False, "split", 0, NW)
    _etype(wh_wt, wt_src, wt_dst, wt_w, _NCH_T, False, "core0", NW, NT)
    _etype(wh_tt, tt_src, tt_dst, tt_w, _NCH_T, True, "core0", NW, NT)
    _etype(wh_wd, wd_src, wd_dst, wd_w, _NCH_D, False, "core1", NW + NT, ND)
    _etype(wh_td, td_src, td_dst, td_w, _NCH_D, True, "core1", NW + NT, ND)


def _sc_aggregate(wh_ww, wh_wt, wh_wd, wh_td, wh_tt, edges):
    (ww_src, ww_dst, ww_w, wt_src, wt_dst, wt_w, wd_src, wd_dst, wd_w,
     td_src, td_dst, td_w, tt_src, tt_dst, tt_w) = edges
    f32 = jnp.float32
    i32 = jnp.int32
    mesh = plsc.VectorSubcoreMesh(core_axis_name="c", subcore_axis_name="s")
    run = pl.kernel(
        _sc_agg_body,
        mesh=mesh,
        compiler_params=pltpu.CompilerParams(needs_layout_passes=False),
        out_type=[
            jax.ShapeDtypeStruct((NW + NT + ND, D), f32),
        ],
        scratch_types=[
            pltpu.VMEM_SHARED((_C + 8, D), f32),
            pltpu.VMEM_SHARED((_C + 8,), f32),
            pltpu.VMEM((_SB,), i32),
            pltpu.VMEM((_SB,), i32),
            pltpu.VMEM((_SB,), f32),
            pltpu.VMEM((_SB,), i32),
            pltpu.VMEM((_SB,), i32),
            pltpu.VMEM((_SB,), f32),
            pltpu.VMEM((_STG,), i32),
            pltpu.VMEM((_STG,), i32),
            pltpu.VMEM((_STG,), f32),
            pltpu.VMEM((128,), i32),
            pltpu.VMEM((128,), i32),
            pltpu.VMEM((128,), f32),
            pltpu.VMEM((128,), i32),
            pltpu.VMEM((128,), i32),
            pltpu.VMEM((128,), f32),
            pltpu.VMEM((128, D), f32),
            pltpu.VMEM((128, D), f32),
            pltpu.VMEM((128,), f32),
            pltpu.VMEM((128,), f32),
            pltpu.VMEM((_C // _NSUB,), f32),
            pltpu.SemaphoreType.DMA,
            pltpu.SemaphoreType.DMA,
            pltpu.SemaphoreType.DMA,
            pltpu.SemaphoreType.DMA,
            pltpu.SemaphoreType.DMA,
            pltpu.SemaphoreType.DMA,
        ],
    )
    return run(wh_ww, wh_wt, wh_wd, wh_td, wh_tt,
               ww_src, ww_dst, ww_w, wt_src, wt_dst, wt_w,
               wd_src, wd_dst, wd_w, td_src, td_dst, td_w,
               tt_src, tt_dst, tt_w)


def kernel(feat_word, feat_topic, effect, ww_w, wt_w, wd_w, td_w, tt_w,
           W_ww, b_ww, W_wt, b_wt, W_wd, b_wd, W_td, b_td, W_tt, b_tt,
           W_cau, W_noi,
           ww_src, ww_dst, wt_src, wt_dst, wd_src, wd_dst,
           td_src, td_dst, tt_src, tt_dst):
    wh_ww, wh_wt, wh_wd = _word_proj(feat_word, W_ww, W_wt, W_wd,
                                     b_ww, b_wt, b_wd)
    wh_td, wh_tt = _topic_proj(feat_topic, effect, W_td, W_tt, W_cau, W_noi,
                               b_td, b_tt)
    flat = []
    for t in (_pad_edges(ww_src, ww_dst, ww_w),
              _pad_edges(wt_src, wt_dst, wt_w),
              _pad_edges(wd_src, wd_dst, wd_w),
              _pad_edges(td_src, td_dst, td_w),
              _pad_edges(tt_src, tt_dst, tt_w)):
        flat.extend(t)
    (out,) = _sc_aggregate(wh_ww, wh_wt, wh_wd, wh_td, wh_tt, tuple(flat))
    return out
